# Initial kernel scaffold; baseline (speedup 1.0000x reference)
import jax
import jax.numpy as jnp
from jax.experimental import pallas as pl


def kernel(x, Adj_data, edge_index, W1, a_src1, a_dst1, b1, W2, a_src2, a_dst2, b2, W3, a_src3, a_dst3, b3, W4, a_src4, a_dst4, b4, alpha1, alpha2, drug_sim, target_sim):
    raise NotImplementedError("write your pallas kernel here")


# speedup vs baseline: 9.0115x; 9.0115x over previous
import functools

import jax
import jax.numpy as jnp
from jax import lax
from jax.experimental import pallas as pl
from jax.experimental.pallas import tpu as pltpu

N_DRUG = 1024
N_TARGET = 1024
N = N_DRUG + N_TARGET
E = 65536
EFULL = E + N
HEADS = 2
F = 32
D = HEADS * F
GAMMAS = (0.01, 0.01, 0.01, 0.01)

_P = jax.lax.Precision.HIGHEST


def _dot(a, b, dims=(((1,), (0,)), ((), ()))):
    return lax.dot_general(a, b, dims, precision=_P,
                           preferred_element_type=jnp.float32)


def _proj1_body(x_ref, w_ref, aa_ref, h_ref, sc_ref):
    h = _dot(x_ref[...], w_ref[...])
    h_ref[...] = h
    sc_ref[...] = _dot(h, aa_ref[...])


def _proj1(x, W1, AA):
    blk = 256
    return pl.pallas_call(
        _proj1_body,
        grid=(N // blk,),
        in_specs=[
            pl.BlockSpec((blk, N), lambda i: (i, 0)),
            pl.BlockSpec((N, D), lambda i: (0, 0)),
            pl.BlockSpec((D, 4), lambda i: (0, 0)),
        ],
        out_specs=[
            pl.BlockSpec((blk, D), lambda i: (i, 0)),
            pl.BlockSpec((blk, 4), lambda i: (i, 0)),
        ],
        out_shape=[
            jax.ShapeDtypeStruct((N, D), jnp.float32),
            jax.ShapeDtypeStruct((N, 4), jnp.float32),
        ],
    )(x, W1, AA)


def _combine_body(num_ref, den_ref, r_ref, b_ref, w_ref, aa_ref,
                  h_ref, sc_ref, ynd_ref, ynt_ref, mud_ref, mut_ref,
                  *, project):
    num = num_ref[0] + num_ref[1]
    den = den_ref[0] + den_ref[1]
    recip = 1.0 / (den.reshape(HEADS, N) + 1e-16)
    scale = lax.dot_general(recip, r_ref[...], (((0,), (0,)), ((), ())),
                            precision=_P, preferred_element_type=jnp.float32)
    H = jnp.maximum(num * scale + b_ref[...], 0.0)
    if project:
        h = _dot(H, w_ref[...])
        h_ref[...] = h
        sc_ref[...] = _dot(h, aa_ref[...])
    else:
        h_ref[...] = H
        sc_ref[...] = jnp.zeros_like(sc_ref)
    yd = H[:N_DRUG]
    yt = H[N_DRUG:]
    for y, yn_ref, mu_ref in ((yd, ynd_ref, mud_ref), (yt, ynt_ref, mut_ref)):
        mn = jnp.min(y, axis=1, keepdims=True)
        mx = jnp.max(y, axis=1, keepdims=True)
        yn = (y - mn) / (mx - mn + 1e-12)
        yn_ref[...] = yn
        mu_ref[...] = jnp.sum(yn * yn).reshape(1, 1) / y.shape[0]


def _combine(num2, den2, Rsel, b_row, Wn, AAn, project):
    full = lambda shape: pl.BlockSpec(shape, lambda: tuple(0 for _ in shape))
    out_shapes = [
        jax.ShapeDtypeStruct((N, D), jnp.float32),
        jax.ShapeDtypeStruct((N, 4), jnp.float32),
        jax.ShapeDtypeStruct((N_DRUG, D), jnp.float32),
        jax.ShapeDtypeStruct((N_TARGET, D), jnp.float32),
        jax.ShapeDtypeStruct((1, 1), jnp.float32),
        jax.ShapeDtypeStruct((1, 1), jnp.float32),
    ]
    out_specs = [full((N, D)), full((N, 4)), full((N_DRUG, D)),
                 full((N_TARGET, D)), full((1, 1)), full((1, 1))]
    res = pl.pallas_call(
        functools.partial(_combine_body, project=project),
        in_specs=[full((2, N, D)), full((2, HEADS * N)), full((HEADS, D)),
                  full((1, D)), full((D, D)), full((D, 4))],
        out_specs=out_specs,
        out_shape=out_shapes,
    )(num2, den2, Rsel, b_row, Wn, AAn)
    return res


def _gip_body(acc_ref, yt_ref, ynf_ref, mu_ref, o_ref, *, gamma, ps, acc_scale):
    yt = yt_ref[...]
    ynf = ynf_ref[...]
    kt = lax.dot_general(yt, ynf, (((1,), (1,)), ((), ())),
                         precision=_P, preferred_element_type=jnp.float32)
    sqni = jnp.sum(yt * yt, axis=1, keepdims=True)
    ones = jnp.ones((1, D), jnp.float32)
    sqnj = lax.dot_general(ones, ynf * ynf, (((1,), (1,)), ((), ())),
                           precision=_P, preferred_element_type=jnp.float32)
    inv_mu = 1.0 / mu_ref[0, 0]
    d = (sqni + sqnj - 2.0 * kt) * inv_mu
    o_ref[...] = acc_ref[...] * acc_scale + ps * jnp.exp(-d * gamma)


def _gip_accum(acc, yn, mu, gamma, ps, acc_scale):
    n = yn.shape[0]
    blk = 256
    return pl.pallas_call(
        functools.partial(_gip_body, gamma=gamma, ps=ps, acc_scale=acc_scale),
        grid=(n // blk,),
        in_specs=[
            pl.BlockSpec((blk, n), lambda i: (i, 0)),
            pl.BlockSpec((blk, D), lambda i: (i, 0)),
            pl.BlockSpec((n, D), lambda i: (0, 0)),
            pl.BlockSpec((1, 1), lambda i: (0, 0)),
        ],
        out_specs=pl.BlockSpec((blk, n), lambda i: (i, 0)),
        out_shape=jax.ShapeDtypeStruct((n, n), jnp.float32),
        input_output_aliases={0: 0},
    )(acc, yn, yn, mu)


def _diag_body(kd_ref, kt_ref, dd_ref, dt_ref, mn_ref, mn_acc):
    j = pl.program_id(0)
    blk = kd_ref.shape[1]
    n = kd_ref.shape[0]
    rows = lax.broadcasted_iota(jnp.int32, (n, blk), 0)
    cols = lax.broadcasted_iota(jnp.int32, (n, blk), 1)
    mask = rows == cols + j * blk

    @pl.when(j == 0)
    def _init():
        mn_acc[0] = jnp.inf
        mn_acc[1] = jnp.inf

    for k_ref, d_ref, slot in ((kd_ref, dd_ref, 0), (kt_ref, dt_ref, 1)):
        k = jnp.abs(k_ref[...])
        d_ref[...] = jnp.sum(jnp.where(mask, k, 0.0), axis=0, keepdims=True)
        pos = jnp.where(k > 0, k, jnp.inf)
        mn_acc[slot] = jnp.minimum(mn_acc[slot], jnp.min(pos))
    mn_ref[0, 0] = mn_acc[0]
    mn_ref[0, 1] = mn_acc[1]


def _diag_minpos(Kd, Kt):
    blk = 256
    return pl.pallas_call(
        _diag_body,
        grid=(N_DRUG // blk,),
        in_specs=[
            pl.BlockSpec((N_DRUG, blk), lambda j: (0, j)),
            pl.BlockSpec((N_TARGET, blk), lambda j: (0, j)),
        ],
        out_specs=[
            pl.BlockSpec((1, blk), lambda j: (0, j)),
            pl.BlockSpec((1, blk), lambda j: (0, j)),
            pl.BlockSpec((1, 2), lambda j: (0, 0), memory_space=pltpu.SMEM),
        ],
        out_shape=[
            jax.ShapeDtypeStruct((1, N_DRUG), jnp.float32),
            jax.ShapeDtypeStruct((1, N_TARGET), jnp.float32),
            jax.ShapeDtypeStruct((1, 2), jnp.float32),
        ],
        scratch_shapes=[pltpu.SMEM((2,), jnp.float32)],
    )(Kd, Kt)


def _final_body(kd_ref, kt_ref, a1_ref, a2_ref, dd_ref, dt_ref, mn_ref, o_ref):
    mnd = mn_ref[0, 0]
    mnt = mn_ref[0, 1]
    dd = dd_ref[...]
    dt = dt_ref[...]
    sd = 1.0 / jnp.where(dd == 0, mnd, dd)
    st = 1.0 / jnp.where(dt == 0, mnt, dt)
    kd = jnp.abs(kd_ref[...])
    kd = jnp.where(kd == 0, mnd, kd) * sd
    kt = jnp.abs(kt_ref[...])
    kt = jnp.where(kt == 0, mnt, kt) * st
    t1 = _dot(kd, a1_ref[...])
    t2 = lax.dot_general(a2_ref[...], kt, (((0,), (1,)), ((), ())),
                         precision=_P, preferred_element_type=jnp.float32)
    o_ref[...] = 0.5 * (t1 + t2)


def _final(Kd, Kt, a1, a2, dd, dt, mn):
    blk = 256
    return pl.pallas_call(
        _final_body,
        grid=(N_DRUG // blk,),
        in_specs=[
            pl.BlockSpec((blk, N_DRUG), lambda i: (i, 0)),
            pl.BlockSpec((N_TARGET, N_TARGET), lambda i: (0, 0)),
            pl.BlockSpec((N_DRUG, N_TARGET), lambda i: (0, 0)),
            pl.BlockSpec((N_TARGET, blk), lambda i: (0, i)),
            pl.BlockSpec((1, N_DRUG), lambda i: (0, 0)),
            pl.BlockSpec((1, N_TARGET), lambda i: (0, 0)),
            pl.BlockSpec((1, 2), lambda i: (0, 0), memory_space=pltpu.SMEM),
        ],
        out_specs=pl.BlockSpec((blk, N_TARGET), lambda i: (i, 0)),
        out_shape=jax.ShapeDtypeStruct((N_DRUG, N_TARGET), jnp.float32),
    )(Kd, Kt, a1, a2, dd, dt, mn)


def _edges_placeholder(h, sc, srcfull, dstfull):
    asrc = sc[:, 0:2]
    adst = sc[:, 2:4]
    e = asrc[srcfull] + adst[dstfull]
    e = jnp.where(e > 0, e, 0.2 * e)
    ex = jnp.exp(e)
    den = jax.ops.segment_sum(ex, dstfull, num_segments=N)
    wexp = jnp.repeat(ex, F, axis=1)
    num = jax.ops.segment_sum(h[srcfull] * wexp, dstfull, num_segments=N)
    num2 = jnp.stack([num, jnp.zeros_like(num)])
    denf = den.T.reshape(-1)
    den2 = jnp.stack([denf, jnp.zeros_like(denf)])
    return num2, den2


def kernel(x, Adj_data, edge_index, W1, a_src1, a_dst1, b1, W2, a_src2,
           a_dst2, b2, W3, a_src3, a_dst3, b3, W4, a_src4, a_dst4, b4,
           alpha1, alpha2, drug_sim, target_sim):
    del Adj_data

    loop = jnp.arange(N, dtype=edge_index.dtype)
    srcfull = jnp.concatenate([edge_index[0], loop])
    dstfull = jnp.concatenate([edge_index[1], loop])

    Rsel = jnp.repeat(jnp.eye(HEADS, dtype=jnp.float32), F, axis=1)

    def make_aa(a_src, a_dst):
        z = jnp.zeros((F,), jnp.float32)
        c0 = jnp.concatenate([a_src[0], z])
        c1 = jnp.concatenate([z, a_src[1]])
        c2 = jnp.concatenate([a_dst[0], z])
        c3 = jnp.concatenate([z, a_dst[1]])
        return jnp.stack([c0, c1, c2, c3], axis=1)

    AAs = [make_aa(a_src1, a_dst1), make_aa(a_src2, a_dst2),
           make_aa(a_src3, a_dst3), make_aa(a_src4, a_dst4)]
    Ws = [W1, W2, W3, W4]
    bs = [b1.reshape(1, D), b2.reshape(1, D), b3.reshape(1, D),
          b4.reshape(1, D)]

    drug_ps = (1.0, 0.5, 0.333, 0.25)
    target_ps = (0.2, 0.2, 0.2, 0.2)

    h, sc = _proj1(x, W1, AAs[0])

    kd_acc = drug_sim
    kt_acc = target_sim
    dummyW = jnp.zeros((D, D), jnp.float32)
    dummyAA = jnp.zeros((D, 4), jnp.float32)
    for layer in range(4):
        num2, den2 = _edges_placeholder(h, sc, srcfull, dstfull)
        last = layer == 3
        Wn = Ws[layer + 1] if not last else dummyW
        AAn = AAs[layer + 1] if not last else dummyAA
        h, sc, ynd, ynt, mud, mut = _combine(
            num2, den2, Rsel, bs[layer], Wn, AAn, project=not last)
        acc_scale = 0.2 if layer == 0 else 1.0
        kd_acc = _gip_accum(kd_acc, ynd, mud, GAMMAS[layer],
                            drug_ps[layer], acc_scale)
        kt_acc = _gip_accum(kt_acc, ynt, mut, GAMMAS[layer],
                            target_ps[layer], acc_scale)

    dd, dt, mn = _diag_minpos(kd_acc, kt_acc)
    return _final(kd_acc, kt_acc, alpha1, alpha2, dd, dt, mn)


# speedup vs baseline: 76.3534x; 8.4729x over previous
import functools

import jax
import jax.numpy as jnp
from jax import lax
from jax.experimental import pallas as pl
from jax.experimental.pallas import tpu as pltpu
from jax.experimental.pallas import tpu_sc as plsc

N_DRUG = 1024
N_TARGET = 1024
N = N_DRUG + N_TARGET
E = 65536
EFULL = E + N
HEADS = 2
F = 32
D = HEADS * F
DP = 128
GAMMAS = (0.01, 0.01, 0.01, 0.01)

_P = jax.lax.Precision.HIGHEST


def _dot(a, b, dims=(((1,), (0,)), ((), ()))):
    return lax.dot_general(a, b, dims, precision=_P,
                           preferred_element_type=jnp.float32)


def _proj1_body(x_ref, w_ref, aa_ref, h_ref, sc_ref):
    h = _dot(x_ref[...], w_ref[...])
    h_ref[...] = jnp.concatenate(
        [h, jnp.zeros((h.shape[0], DP - D), jnp.float32)], axis=1)
    sc_ref[...] = _dot(h, aa_ref[...])


def _proj1(x, W1, AA):
    blk = 256
    return pl.pallas_call(
        _proj1_body,
        grid=(N // blk,),
        in_specs=[
            pl.BlockSpec((blk, N), lambda i: (i, 0)),
            pl.BlockSpec((N, D), lambda i: (0, 0)),
            pl.BlockSpec((D, 4), lambda i: (0, 0)),
        ],
        out_specs=[
            pl.BlockSpec((blk, DP), lambda i: (i, 0)),
            pl.BlockSpec((blk, 4), lambda i: (i, 0)),
        ],
        out_shape=[
            jax.ShapeDtypeStruct((N, DP), jnp.float32),
            jax.ShapeDtypeStruct((N, 4), jnp.float32),
        ],
    )(x, W1, AA)


def _combine_body(num_ref, r_ref, b_ref, w_ref, aa_ref,
                  h_ref, sc_ref, ynd_ref, ynt_ref, mud_ref, mut_ref,
                  *, project):
    acc = num_ref[0] + num_ref[1]
    num = acc[:, :D]
    den = acc[:, D:D + HEADS]
    recip = 1.0 / (den + 1e-16)
    scale = _dot(recip, r_ref[...])
    H = jnp.maximum(num * scale + b_ref[...], 0.0)
    pad = jnp.zeros((N, DP - D), jnp.float32)
    if project:
        h = _dot(H, w_ref[...])
        h_ref[...] = jnp.concatenate([h, pad], axis=1)
        sc_ref[...] = _dot(h, aa_ref[...])
    else:
        h_ref[...] = jnp.concatenate([H, pad], axis=1)
        sc_ref[...] = jnp.zeros_like(sc_ref)
    yd = H[:N_DRUG]
    yt = H[N_DRUG:]
    for y, yn_ref, mu_ref in ((yd, ynd_ref, mud_ref), (yt, ynt_ref, mut_ref)):
        mn = jnp.min(y, axis=1, keepdims=True)
        mx = jnp.max(y, axis=1, keepdims=True)
        yn = (y - mn) / (mx - mn + 1e-12)
        yn_ref[...] = yn
        mu_ref[...] = jnp.sum(yn * yn).reshape(1, 1) / y.shape[0]


def _combine(num2, Rsel, b_row, Wn, AAn, project):
    full = lambda shape: pl.BlockSpec(shape, lambda: tuple(0 for _ in shape))
    out_shapes = [
        jax.ShapeDtypeStruct((N, DP), jnp.float32),
        jax.ShapeDtypeStruct((N, 4), jnp.float32),
        jax.ShapeDtypeStruct((N_DRUG, D), jnp.float32),
        jax.ShapeDtypeStruct((N_TARGET, D), jnp.float32),
        jax.ShapeDtypeStruct((1, 1), jnp.float32),
        jax.ShapeDtypeStruct((1, 1), jnp.float32),
    ]
    out_specs = [full((N, DP)), full((N, 4)), full((N_DRUG, D)),
                 full((N_TARGET, D)), full((1, 1)), full((1, 1))]
    res = pl.pallas_call(
        functools.partial(_combine_body, project=project),
        in_specs=[full((2, N, DP)), full((HEADS, D)),
                  full((1, D)), full((D, D)), full((D, 4))],
        out_specs=out_specs,
        out_shape=out_shapes,
    )(num2, Rsel, b_row, Wn, AAn)
    return res


def _gip_body(acc_ref, yt_ref, ynf_ref, mu_ref, o_ref, *, gamma, ps, acc_scale):
    yt = yt_ref[...]
    ynf = ynf_ref[...]
    kt = lax.dot_general(yt, ynf, (((1,), (1,)), ((), ())),
                         precision=_P, preferred_element_type=jnp.float32)
    sqni = jnp.sum(yt * yt, axis=1, keepdims=True)
    ones = jnp.ones((1, D), jnp.float32)
    sqnj = lax.dot_general(ones, ynf * ynf, (((1,), (1,)), ((), ())),
                           precision=_P, preferred_element_type=jnp.float32)
    inv_mu = 1.0 / mu_ref[0, 0]
    d = (sqni + sqnj - 2.0 * kt) * inv_mu
    o_ref[...] = acc_ref[...] * acc_scale + ps * jnp.exp(-d * gamma)


def _gip_accum(acc, yn, mu, gamma, ps, acc_scale):
    n = yn.shape[0]
    blk = 256
    return pl.pallas_call(
        functools.partial(_gip_body, gamma=gamma, ps=ps, acc_scale=acc_scale),
        grid=(n // blk,),
        in_specs=[
            pl.BlockSpec((blk, n), lambda i: (i, 0)),
            pl.BlockSpec((blk, D), lambda i: (i, 0)),
            pl.BlockSpec((n, D), lambda i: (0, 0)),
            pl.BlockSpec((1, 1), lambda i: (0, 0)),
        ],
        out_specs=pl.BlockSpec((blk, n), lambda i: (i, 0)),
        out_shape=jax.ShapeDtypeStruct((n, n), jnp.float32),
        input_output_aliases={0: 0},
    )(acc, yn, yn, mu)


def _diag_body(kd_ref, kt_ref, dd_ref, dt_ref, mn_ref, mn_acc):
    j = pl.program_id(0)
    blk = kd_ref.shape[1]
    n = kd_ref.shape[0]
    rows = lax.broadcasted_iota(jnp.int32, (n, blk), 0)
    cols = lax.broadcasted_iota(jnp.int32, (n, blk), 1)
    mask = rows == cols + j * blk

    @pl.when(j == 0)
    def _init():
        mn_acc[0] = jnp.inf
        mn_acc[1] = jnp.inf

    for k_ref, d_ref, slot in ((kd_ref, dd_ref, 0), (kt_ref, dt_ref, 1)):
        k = jnp.abs(k_ref[...])
        d_ref[...] = jnp.sum(jnp.where(mask, k, 0.0), axis=0, keepdims=True)
        pos = jnp.where(k > 0, k, jnp.inf)
        mn_acc[slot] = jnp.minimum(mn_acc[slot], jnp.min(pos))
    mn_ref[0, 0] = mn_acc[0]
    mn_ref[0, 1] = mn_acc[1]


def _diag_minpos(Kd, Kt):
    blk = 256
    return pl.pallas_call(
        _diag_body,
        grid=(N_DRUG // blk,),
        in_specs=[
            pl.BlockSpec((N_DRUG, blk), lambda j: (0, j)),
            pl.BlockSpec((N_TARGET, blk), lambda j: (0, j)),
        ],
        out_specs=[
            pl.BlockSpec((1, blk), lambda j: (0, j)),
            pl.BlockSpec((1, blk), lambda j: (0, j)),
            pl.BlockSpec((1, 2), lambda j: (0, 0), memory_space=pltpu.SMEM),
        ],
        out_shape=[
            jax.ShapeDtypeStruct((1, N_DRUG), jnp.float32),
            jax.ShapeDtypeStruct((1, N_TARGET), jnp.float32),
            jax.ShapeDtypeStruct((1, 2), jnp.float32),
        ],
        scratch_shapes=[pltpu.SMEM((2,), jnp.float32)],
    )(Kd, Kt)


def _final_body(kd_ref, kt_ref, a1_ref, a2_ref, dd_ref, dt_ref, mn_ref, o_ref):
    mnd = mn_ref[0, 0]
    mnt = mn_ref[0, 1]
    dd = dd_ref[...]
    dt = dt_ref[...]
    sd = 1.0 / jnp.where(dd == 0, mnd, dd)
    st = 1.0 / jnp.where(dt == 0, mnt, dt)
    kd = jnp.abs(kd_ref[...])
    kd = jnp.where(kd == 0, mnd, kd) * sd
    kt = jnp.abs(kt_ref[...])
    kt = jnp.where(kt == 0, mnt, kt) * st
    t1 = _dot(kd, a1_ref[...])
    t2 = lax.dot_general(a2_ref[...], kt, (((0,), (1,)), ((), ())),
                         precision=_P, preferred_element_type=jnp.float32)
    o_ref[...] = 0.5 * (t1 + t2)


def _final(Kd, Kt, a1, a2, dd, dt, mn):
    blk = 256
    return pl.pallas_call(
        _final_body,
        grid=(N_DRUG // blk,),
        in_specs=[
            pl.BlockSpec((blk, N_DRUG), lambda i: (i, 0)),
            pl.BlockSpec((N_TARGET, N_TARGET), lambda i: (0, 0)),
            pl.BlockSpec((N_DRUG, N_TARGET), lambda i: (0, 0)),
            pl.BlockSpec((N_TARGET, blk), lambda i: (0, i)),
            pl.BlockSpec((1, N_DRUG), lambda i: (0, 0)),
            pl.BlockSpec((1, N_TARGET), lambda i: (0, 0)),
            pl.BlockSpec((1, 2), lambda i: (0, 0), memory_space=pltpu.SMEM),
        ],
        out_specs=pl.BlockSpec((blk, N_TARGET), lambda i: (i, 0)),
        out_shape=jax.ShapeDtypeStruct((N_DRUG, N_TARGET), jnp.float32),
    )(Kd, Kt, a1, a2, dd, dt, mn)


NWORK = 32
EW = EFULL // NWORK
CH = 96
NCHUNK = EW // CH
NGROUP = CH // 16


def _edge_body(h_hbm, sc_hbm, src_hbm, dst_hbm, z2_hbm,
               num_hbm,
               sc_tab, src_v, dst_v, ex0_v, ex1_v, rows_v,
               num_sp, sem):
    c = lax.axis_index("c")
    s = lax.axis_index("s")
    wid = s * 2 + c
    base = pl.multiple_of(wid * EW, 8)

    pltpu.sync_copy(sc_hbm, sc_tab)

    nsl = N // 16
    pltpu.sync_copy(z2_hbm.at[pl.ds(s * nsl, nsl)],
                    num_sp.at[pl.ds(s * nsl, nsl)])
    plsc.subcore_barrier()

    def chunk_body(jc, carry):
        off = pl.multiple_of(base + jc * CH, 8)
        pltpu.sync_copy(src_hbm.at[pl.ds(off, CH)], src_v)
        pltpu.sync_copy(dst_hbm.at[pl.ds(off, CH)], dst_v)
        gat = pltpu.async_copy(h_hbm.at[src_v], rows_v, sem)

        def group_body(g, carry2):
            go = g * 16
            sidx = src_v[pl.ds(go, 16)]
            didx = dst_v[pl.ds(go, 16)]
            as0 = plsc.load_gather(sc_tab, [sidx * 4])
            as1 = plsc.load_gather(sc_tab, [sidx * 4 + 1])
            ad0 = plsc.load_gather(sc_tab, [didx * 4 + 2])
            ad1 = plsc.load_gather(sc_tab, [didx * 4 + 3])
            e0 = as0 + ad0
            e1 = as1 + ad1
            e0 = jnp.where(e0 > 0, e0, 0.2 * e0)
            e1 = jnp.where(e1 > 0, e1, 0.2 * e1)
            ex0_v[pl.ds(go, 16)] = jnp.exp(e0)
            ex1_v[pl.ds(go, 16)] = jnp.exp(e1)
            return carry2

        lax.fori_loop(0, NGROUP, group_body, 0)
        gat.wait()
        iot = lax.iota(jnp.int32, 16)

        def edge_body(e, carry2):
            i0 = jnp.full((16,), e, jnp.int32)
            w0 = plsc.load_gather(ex0_v, [i0])
            w1 = plsc.load_gather(ex1_v, [i0])
            r0 = rows_v[e, pl.ds(0, 16)] * w0
            r1 = rows_v[e, pl.ds(16, 16)] * w0
            r2 = rows_v[e, pl.ds(32, 16)] * w1
            r3 = rows_v[e, pl.ds(48, 16)] * w1
            rows_v[e, pl.ds(0, 16)] = r0
            rows_v[e, pl.ds(16, 16)] = r1
            rows_v[e, pl.ds(32, 16)] = r2
            rows_v[e, pl.ds(48, 16)] = r3
            dv = jnp.where(iot == 0, w0, jnp.where(iot == 1, w1, 0.0))
            rows_v[e, pl.ds(D, 16)] = dv
            return carry2

        lax.fori_loop(0, CH, edge_body, 0)

        pltpu.sync_copy(rows_v, num_sp.at[dst_v], add=True)
        return carry

    lax.fori_loop(0, NCHUNK, chunk_body, 0)
    plsc.subcore_barrier()

    pltpu.sync_copy(num_sp.at[pl.ds(s * nsl, nsl)],
                    num_hbm.at[c, pl.ds(s * nsl, nsl)])


def _edges_sc(h, scflat, srcfull, dstfull):
    mesh = plsc.VectorSubcoreMesh(core_axis_name="c", subcore_axis_name="s")
    z2 = jnp.zeros((N, DP), jnp.float32)
    call = pl.kernel(
        _edge_body,
        out_type=jax.ShapeDtypeStruct((2, N, DP), jnp.float32),
        mesh=mesh,
        compiler_params=pltpu.CompilerParams(needs_layout_passes=False),
        scratch_types=[
            pltpu.VMEM((4 * N,), jnp.float32),
            pltpu.VMEM((CH,), jnp.int32),
            pltpu.VMEM((CH,), jnp.int32),
            pltpu.VMEM((CH,), jnp.float32),
            pltpu.VMEM((CH,), jnp.float32),
            pltpu.VMEM((CH, DP), jnp.float32),
            pltpu.VMEM_SHARED((N, DP), jnp.float32),
            pltpu.SemaphoreType.DMA,
        ],
    )
    return call(h, scflat, srcfull, dstfull, z2)


def kernel(x, Adj_data, edge_index, W1, a_src1, a_dst1, b1, W2, a_src2,
           a_dst2, b2, W3, a_src3, a_dst3, b3, W4, a_src4, a_dst4, b4,
           alpha1, alpha2, drug_sim, target_sim):
    del Adj_data

    loop = jnp.arange(N, dtype=edge_index.dtype)
    srcfull = jnp.concatenate([edge_index[0], loop])
    dstfull = jnp.concatenate([edge_index[1], loop])

    Rsel = jnp.repeat(jnp.eye(HEADS, dtype=jnp.float32), F, axis=1)

    def make_aa(a_src, a_dst):
        z = jnp.zeros((F,), jnp.float32)
        c0 = jnp.concatenate([a_src[0], z])
        c1 = jnp.concatenate([z, a_src[1]])
        c2 = jnp.concatenate([a_dst[0], z])
        c3 = jnp.concatenate([z, a_dst[1]])
        return jnp.stack([c0, c1, c2, c3], axis=1)

    AAs = [make_aa(a_src1, a_dst1), make_aa(a_src2, a_dst2),
           make_aa(a_src3, a_dst3), make_aa(a_src4, a_dst4)]
    Ws = [W1, W2, W3, W4]
    bs = [b1.reshape(1, D), b2.reshape(1, D), b3.reshape(1, D),
          b4.reshape(1, D)]

    drug_ps = (1.0, 0.5, 0.333, 0.25)
    target_ps = (0.2, 0.2, 0.2, 0.2)

    h, sc = _proj1(x, W1, AAs[0])

    kd_acc = drug_sim
    kt_acc = target_sim
    dummyW = jnp.zeros((D, D), jnp.float32)
    dummyAA = jnp.zeros((D, 4), jnp.float32)
    for layer in range(4):
        num2 = _edges_sc(h, sc.reshape(-1), srcfull, dstfull)
        last = layer == 3
        Wn = Ws[layer + 1] if not last else dummyW
        AAn = AAs[layer + 1] if not last else dummyAA
        h, sc, ynd, ynt, mud, mut = _combine(
            num2, Rsel, bs[layer], Wn, AAn, project=not last)
        acc_scale = 0.2 if layer == 0 else 1.0
        kd_acc = _gip_accum(kd_acc, ynd, mud, GAMMAS[layer],
                            drug_ps[layer], acc_scale)
        kt_acc = _gip_accum(kt_acc, ynt, mut, GAMMAS[layer],
                            target_ps[layer], acc_scale)

    dd, dt, mn = _diag_minpos(kd_acc, kt_acc)
    return _final(kd_acc, kt_acc, alpha1, alpha2, dd, dt, mn)


# speedup vs baseline: 80.1262x; 1.0494x over previous
import functools

import jax
import jax.numpy as jnp
from jax import lax
from jax.experimental import pallas as pl
from jax.experimental.pallas import tpu as pltpu
from jax.experimental.pallas import tpu_sc as plsc

N_DRUG = 1024
N_TARGET = 1024
N = N_DRUG + N_TARGET
E = 65536
EFULL = E + N
HEADS = 2
F = 32
D = HEADS * F
DP = 128
GAMMAS = (0.01, 0.01, 0.01, 0.01)

_P = jax.lax.Precision.HIGHEST


def _dot(a, b, dims=(((1,), (0,)), ((), ()))):
    return lax.dot_general(a, b, dims, precision=_P,
                           preferred_element_type=jnp.float32)


def _proj1_body(x_ref, w_ref, aa_ref, h_ref, sc_ref):
    h = _dot(x_ref[...], w_ref[...])
    h_ref[...] = jnp.concatenate(
        [h, jnp.zeros((h.shape[0], DP - D), jnp.float32)], axis=1)
    sc_ref[...] = _dot(h, aa_ref[...])


def _proj1(x, W1, AA):
    blk = 256
    return pl.pallas_call(
        _proj1_body,
        grid=(N // blk,),
        in_specs=[
            pl.BlockSpec((blk, N), lambda i: (i, 0)),
            pl.BlockSpec((N, D), lambda i: (0, 0)),
            pl.BlockSpec((D, 4), lambda i: (0, 0)),
        ],
        out_specs=[
            pl.BlockSpec((blk, DP), lambda i: (i, 0)),
            pl.BlockSpec((blk, 4), lambda i: (i, 0)),
        ],
        out_shape=[
            jax.ShapeDtypeStruct((N, DP), jnp.float32),
            jax.ShapeDtypeStruct((N, 4), jnp.float32),
        ],
    )(x, W1, AA)


def _combine_body(num_ref, r_ref, b_ref, w_ref, aa_ref,
                  h_ref, sc_ref, ynd_ref, ynt_ref, mud_ref, mut_ref,
                  *, project):
    acc = num_ref[0] + num_ref[1]
    num = acc[:, :D]
    den = acc[:, D:D + HEADS]
    recip = 1.0 / (den + 1e-16)
    scale = _dot(recip, r_ref[...])
    H = jnp.maximum(num * scale + b_ref[...], 0.0)
    pad = jnp.zeros((N, DP - D), jnp.float32)
    if project:
        h = _dot(H, w_ref[...])
        h_ref[...] = jnp.concatenate([h, pad], axis=1)
        sc_ref[...] = _dot(h, aa_ref[...])
    else:
        h_ref[...] = jnp.concatenate([H, pad], axis=1)
        sc_ref[...] = jnp.zeros_like(sc_ref)
    yd = H[:N_DRUG]
    yt = H[N_DRUG:]
    for y, yn_ref, mu_ref in ((yd, ynd_ref, mud_ref), (yt, ynt_ref, mut_ref)):
        mn = jnp.min(y, axis=1, keepdims=True)
        mx = jnp.max(y, axis=1, keepdims=True)
        yn = (y - mn) / (mx - mn + 1e-12)
        yn_ref[...] = yn
        mu_ref[...] = jnp.sum(yn * yn).reshape(1, 1) / y.shape[0]


def _combine(num2, Rsel, b_row, Wn, AAn, project):
    full = lambda shape: pl.BlockSpec(shape, lambda: tuple(0 for _ in shape))
    out_shapes = [
        jax.ShapeDtypeStruct((N, DP), jnp.float32),
        jax.ShapeDtypeStruct((N, 4), jnp.float32),
        jax.ShapeDtypeStruct((N_DRUG, D), jnp.float32),
        jax.ShapeDtypeStruct((N_TARGET, D), jnp.float32),
        jax.ShapeDtypeStruct((1, 1), jnp.float32),
        jax.ShapeDtypeStruct((1, 1), jnp.float32),
    ]
    out_specs = [full((N, DP)), full((N, 4)), full((N_DRUG, D)),
                 full((N_TARGET, D)), full((1, 1)), full((1, 1))]
    res = pl.pallas_call(
        functools.partial(_combine_body, project=project),
        in_specs=[full((2, N, DP)), full((HEADS, D)),
                  full((1, D)), full((D, D)), full((D, 4))],
        out_specs=out_specs,
        out_shape=out_shapes,
    )(num2, Rsel, b_row, Wn, AAn)
    return res


def _gip_body(acc_ref, yt_ref, ynf_ref, mu_ref, o_ref, *, gamma, ps, acc_scale):
    yt = yt_ref[...]
    ynf = ynf_ref[...]
    kt = lax.dot_general(yt, ynf, (((1,), (1,)), ((), ())),
                         precision=_P, preferred_element_type=jnp.float32)
    sqni = jnp.sum(yt * yt, axis=1, keepdims=True)
    ones = jnp.ones((1, D), jnp.float32)
    sqnj = lax.dot_general(ones, ynf * ynf, (((1,), (1,)), ((), ())),
                           precision=_P, preferred_element_type=jnp.float32)
    inv_mu = 1.0 / mu_ref[0, 0]
    d = (sqni + sqnj - 2.0 * kt) * inv_mu
    o_ref[...] = acc_ref[...] * acc_scale + ps * jnp.exp(-d * gamma)


def _gip_accum(acc, yn, mu, gamma, ps, acc_scale):
    n = yn.shape[0]
    blk = 256
    return pl.pallas_call(
        functools.partial(_gip_body, gamma=gamma, ps=ps, acc_scale=acc_scale),
        grid=(n // blk,),
        in_specs=[
            pl.BlockSpec((blk, n), lambda i: (i, 0)),
            pl.BlockSpec((blk, D), lambda i: (i, 0)),
            pl.BlockSpec((n, D), lambda i: (0, 0)),
            pl.BlockSpec((1, 1), lambda i: (0, 0)),
        ],
        out_specs=pl.BlockSpec((blk, n), lambda i: (i, 0)),
        out_shape=jax.ShapeDtypeStruct((n, n), jnp.float32),
        input_output_aliases={0: 0},
    )(acc, yn, yn, mu)


def _diag_body(kd_ref, kt_ref, dd_ref, dt_ref, mn_ref, mn_acc):
    j = pl.program_id(0)
    blk = kd_ref.shape[1]
    n = kd_ref.shape[0]
    rows = lax.broadcasted_iota(jnp.int32, (n, blk), 0)
    cols = lax.broadcasted_iota(jnp.int32, (n, blk), 1)
    mask = rows == cols + j * blk

    @pl.when(j == 0)
    def _init():
        mn_acc[0] = jnp.inf
        mn_acc[1] = jnp.inf

    for k_ref, d_ref, slot in ((kd_ref, dd_ref, 0), (kt_ref, dt_ref, 1)):
        k = jnp.abs(k_ref[...])
        d_ref[...] = jnp.sum(jnp.where(mask, k, 0.0), axis=0, keepdims=True)
        pos = jnp.where(k > 0, k, jnp.inf)
        mn_acc[slot] = jnp.minimum(mn_acc[slot], jnp.min(pos))
    mn_ref[0, 0] = mn_acc[0]
    mn_ref[0, 1] = mn_acc[1]


def _diag_minpos(Kd, Kt):
    blk = 256
    return pl.pallas_call(
        _diag_body,
        grid=(N_DRUG // blk,),
        in_specs=[
            pl.BlockSpec((N_DRUG, blk), lambda j: (0, j)),
            pl.BlockSpec((N_TARGET, blk), lambda j: (0, j)),
        ],
        out_specs=[
            pl.BlockSpec((1, blk), lambda j: (0, j)),
            pl.BlockSpec((1, blk), lambda j: (0, j)),
            pl.BlockSpec((1, 2), lambda j: (0, 0), memory_space=pltpu.SMEM),
        ],
        out_shape=[
            jax.ShapeDtypeStruct((1, N_DRUG), jnp.float32),
            jax.ShapeDtypeStruct((1, N_TARGET), jnp.float32),
            jax.ShapeDtypeStruct((1, 2), jnp.float32),
        ],
        scratch_shapes=[pltpu.SMEM((2,), jnp.float32)],
    )(Kd, Kt)


def _final_body(kd_ref, kt_ref, a1_ref, a2_ref, dd_ref, dt_ref, mn_ref, o_ref):
    mnd = mn_ref[0, 0]
    mnt = mn_ref[0, 1]
    dd = dd_ref[...]
    dt = dt_ref[...]
    sd = 1.0 / jnp.where(dd == 0, mnd, dd)
    st = 1.0 / jnp.where(dt == 0, mnt, dt)
    kd = jnp.abs(kd_ref[...])
    kd = jnp.where(kd == 0, mnd, kd) * sd
    kt = jnp.abs(kt_ref[...])
    kt = jnp.where(kt == 0, mnt, kt) * st
    t1 = _dot(kd, a1_ref[...])
    t2 = lax.dot_general(a2_ref[...], kt, (((0,), (1,)), ((), ())),
                         precision=_P, preferred_element_type=jnp.float32)
    o_ref[...] = 0.5 * (t1 + t2)


def _final(Kd, Kt, a1, a2, dd, dt, mn):
    blk = 256
    return pl.pallas_call(
        _final_body,
        grid=(N_DRUG // blk,),
        in_specs=[
            pl.BlockSpec((blk, N_DRUG), lambda i: (i, 0)),
            pl.BlockSpec((N_TARGET, N_TARGET), lambda i: (0, 0)),
            pl.BlockSpec((N_DRUG, N_TARGET), lambda i: (0, 0)),
            pl.BlockSpec((N_TARGET, blk), lambda i: (0, i)),
            pl.BlockSpec((1, N_DRUG), lambda i: (0, 0)),
            pl.BlockSpec((1, N_TARGET), lambda i: (0, 0)),
            pl.BlockSpec((1, 2), lambda i: (0, 0), memory_space=pltpu.SMEM),
        ],
        out_specs=pl.BlockSpec((blk, N_TARGET), lambda i: (i, 0)),
        out_shape=jax.ShapeDtypeStruct((N_DRUG, N_TARGET), jnp.float32),
    )(Kd, Kt, a1, a2, dd, dt, mn)


NWORK = 32
EW = EFULL // NWORK
CH = 96
NCHUNK = EW // CH
NGROUP = CH // 16


def _edge_body(h_hbm, sc_hbm, src_hbm, dst_hbm, z2_hbm,
               num_hbm,
               sc_tab, src_v, dst_v, ex0_v, ex1_v, rows_v,
               num_sp, sem):
    c = lax.axis_index("c")
    s = lax.axis_index("s")
    wid = s * 2 + c
    base = pl.multiple_of(wid * EW, 8)

    pltpu.sync_copy(sc_hbm, sc_tab)

    nsl = N // 16
    pltpu.sync_copy(z2_hbm.at[pl.ds(s * nsl, nsl)],
                    num_sp.at[pl.ds(s * nsl, nsl)])
    plsc.subcore_barrier()

    def chunk_body(jc, carry):
        off = pl.multiple_of(base + jc * CH, 8)
        pltpu.sync_copy(src_hbm.at[pl.ds(off, CH)], src_v)
        pltpu.sync_copy(dst_hbm.at[pl.ds(off, CH)], dst_v)
        gat = pltpu.async_copy(h_hbm.at[src_v], rows_v, sem)

        def group_body(g, carry2):
            go = g * 16
            sidx = src_v[pl.ds(go, 16)]
            didx = dst_v[pl.ds(go, 16)]
            as0 = plsc.load_gather(sc_tab, [sidx * 4])
            as1 = plsc.load_gather(sc_tab, [sidx * 4 + 1])
            ad0 = plsc.load_gather(sc_tab, [didx * 4 + 2])
            ad1 = plsc.load_gather(sc_tab, [didx * 4 + 3])
            e0 = as0 + ad0
            e1 = as1 + ad1
            e0 = jnp.where(e0 > 0, e0, 0.2 * e0)
            e1 = jnp.where(e1 > 0, e1, 0.2 * e1)
            ex0_v[pl.ds(go, 16)] = jnp.exp(e0)
            ex1_v[pl.ds(go, 16)] = jnp.exp(e1)
            return carry2

        lax.fori_loop(0, NGROUP, group_body, 0)
        gat.wait()
        iot = lax.iota(jnp.int32, 16)

        def den_body(g, carry2):
            go = g * 16
            ridx = go + iot
            plsc.store_scatter(rows_v, [ridx, jnp.full((16,), D, jnp.int32)],
                               ex0_v[pl.ds(go, 16)])
            plsc.store_scatter(rows_v,
                               [ridx, jnp.full((16,), D + 1, jnp.int32)],
                               ex1_v[pl.ds(go, 16)])
            return carry2

        lax.fori_loop(0, NGROUP, den_body, 0)

        @plsc.parallel_loop(0, CH, step=1, unroll=4)
        def edge_body(e):
            i0 = jnp.full((16,), e, jnp.int32)
            w0 = plsc.load_gather(ex0_v, [i0])
            w1 = plsc.load_gather(ex1_v, [i0])
            rows_v[e, pl.ds(0, 16)] = rows_v[e, pl.ds(0, 16)] * w0
            rows_v[e, pl.ds(16, 16)] = rows_v[e, pl.ds(16, 16)] * w0
            rows_v[e, pl.ds(32, 16)] = rows_v[e, pl.ds(32, 16)] * w1
            rows_v[e, pl.ds(48, 16)] = rows_v[e, pl.ds(48, 16)] * w1

        pltpu.sync_copy(rows_v, num_sp.at[dst_v], add=True)
        return carry

    lax.fori_loop(0, NCHUNK, chunk_body, 0)
    plsc.subcore_barrier()

    pltpu.sync_copy(num_sp.at[pl.ds(s * nsl, nsl)],
                    num_hbm.at[c, pl.ds(s * nsl, nsl)])


def _edges_sc(h, scflat, srcfull, dstfull):
    mesh = plsc.VectorSubcoreMesh(core_axis_name="c", subcore_axis_name="s")
    z2 = jnp.zeros((N, DP), jnp.float32)
    call = pl.kernel(
        _edge_body,
        out_type=jax.ShapeDtypeStruct((2, N, DP), jnp.float32),
        mesh=mesh,
        compiler_params=pltpu.CompilerParams(needs_layout_passes=False),
        scratch_types=[
            pltpu.VMEM((4 * N,), jnp.float32),
            pltpu.VMEM((CH,), jnp.int32),
            pltpu.VMEM((CH,), jnp.int32),
            pltpu.VMEM((CH,), jnp.float32),
            pltpu.VMEM((CH,), jnp.float32),
            pltpu.VMEM((CH, DP), jnp.float32),
            pltpu.VMEM_SHARED((N, DP), jnp.float32),
            pltpu.SemaphoreType.DMA,
        ],
    )
    return call(h, scflat, srcfull, dstfull, z2)


def kernel(x, Adj_data, edge_index, W1, a_src1, a_dst1, b1, W2, a_src2,
           a_dst2, b2, W3, a_src3, a_dst3, b3, W4, a_src4, a_dst4, b4,
           alpha1, alpha2, drug_sim, target_sim):
    del Adj_data

    loop = jnp.arange(N, dtype=edge_index.dtype)
    srcfull = jnp.concatenate([edge_index[0], loop])
    dstfull = jnp.concatenate([edge_index[1], loop])

    Rsel = jnp.repeat(jnp.eye(HEADS, dtype=jnp.float32), F, axis=1)

    def make_aa(a_src, a_dst):
        z = jnp.zeros((F,), jnp.float32)
        c0 = jnp.concatenate([a_src[0], z])
        c1 = jnp.concatenate([z, a_src[1]])
        c2 = jnp.concatenate([a_dst[0], z])
        c3 = jnp.concatenate([z, a_dst[1]])
        return jnp.stack([c0, c1, c2, c3], axis=1)

    AAs = [make_aa(a_src1, a_dst1), make_aa(a_src2, a_dst2),
           make_aa(a_src3, a_dst3), make_aa(a_src4, a_dst4)]
    Ws = [W1, W2, W3, W4]
    bs = [b1.reshape(1, D), b2.reshape(1, D), b3.reshape(1, D),
          b4.reshape(1, D)]

    drug_ps = (1.0, 0.5, 0.333, 0.25)
    target_ps = (0.2, 0.2, 0.2, 0.2)

    h, sc = _proj1(x, W1, AAs[0])

    kd_acc = drug_sim
    kt_acc = target_sim
    dummyW = jnp.zeros((D, D), jnp.float32)
    dummyAA = jnp.zeros((D, 4), jnp.float32)
    for layer in range(4):
        num2 = _edges_sc(h, sc.reshape(-1), srcfull, dstfull)
        last = layer == 3
        Wn = Ws[layer + 1] if not last else dummyW
        AAn = AAs[layer + 1] if not last else dummyAA
        h, sc, ynd, ynt, mud, mut = _combine(
            num2, Rsel, bs[layer], Wn, AAn, project=not last)
        acc_scale = 0.2 if layer == 0 else 1.0
        kd_acc = _gip_accum(kd_acc, ynd, mud, GAMMAS[layer],
                            drug_ps[layer], acc_scale)
        kt_acc = _gip_accum(kt_acc, ynt, mut, GAMMAS[layer],
                            target_ps[layer], acc_scale)

    dd, dt, mn = _diag_minpos(kd_acc, kt_acc)
    return _final(kd_acc, kt_acc, alpha1, alpha2, dd, dt, mn)


# speedup vs baseline: 107.9164x; 1.3468x over previous
import functools

import jax
import jax.numpy as jnp
from jax import lax
from jax.experimental import pallas as pl
from jax.experimental.pallas import tpu as pltpu
from jax.experimental.pallas import tpu_sc as plsc

N_DRUG = 1024
N_TARGET = 1024
N = N_DRUG + N_TARGET
E = 65536
EFULL = E + N
HEADS = 2
F = 32
D = HEADS * F
DP = 128
GAMMAS = (0.01, 0.01, 0.01, 0.01)

_P = jax.lax.Precision.HIGHEST


def _dot(a, b, dims=(((1,), (0,)), ((), ()))):
    return lax.dot_general(a, b, dims, precision=_P,
                           preferred_element_type=jnp.float32)


def _proj1_body(x_ref, w_ref, aa_ref, h_ref, sc_ref):
    h = _dot(x_ref[...], w_ref[...])
    h_ref[...] = jnp.concatenate(
        [h, jnp.zeros((h.shape[0], DP - D), jnp.float32)], axis=1)
    sc_ref[...] = _dot(h, aa_ref[...])


def _proj1(x, W1, AA):
    blk = 256
    return pl.pallas_call(
        _proj1_body,
        grid=(N // blk,),
        in_specs=[
            pl.BlockSpec((blk, N), lambda i: (i, 0)),
            pl.BlockSpec((N, D), lambda i: (0, 0)),
            pl.BlockSpec((D, 4), lambda i: (0, 0)),
        ],
        out_specs=[
            pl.BlockSpec((blk, DP), lambda i: (i, 0)),
            pl.BlockSpec((blk, 4), lambda i: (i, 0)),
        ],
        out_shape=[
            jax.ShapeDtypeStruct((N, DP), jnp.float32),
            jax.ShapeDtypeStruct((N, 4), jnp.float32),
        ],
    )(x, W1, AA)


def _combine_body(num_ref, r_ref, b_ref, w_ref, aa_ref,
                  h_ref, sc_ref, ynd_ref, ynt_ref, mud_ref, mut_ref,
                  *, project):
    acc = num_ref[0] + num_ref[1]
    num = acc[:, :D]
    den = acc[:, D:D + HEADS]
    recip = 1.0 / (den + 1e-16)
    scale = _dot(recip, r_ref[...])
    H = jnp.maximum(num * scale + b_ref[...], 0.0)
    pad = jnp.zeros((N, DP - D), jnp.float32)
    if project:
        h = _dot(H, w_ref[...])
        h_ref[...] = jnp.concatenate([h, pad], axis=1)
        sc_ref[...] = _dot(h, aa_ref[...])
    else:
        h_ref[...] = jnp.concatenate([H, pad], axis=1)
        sc_ref[...] = jnp.zeros_like(sc_ref)
    yd = H[:N_DRUG]
    yt = H[N_DRUG:]
    for y, yn_ref, mu_ref in ((yd, ynd_ref, mud_ref), (yt, ynt_ref, mut_ref)):
        mn = jnp.min(y, axis=1, keepdims=True)
        mx = jnp.max(y, axis=1, keepdims=True)
        yn = (y - mn) / (mx - mn + 1e-12)
        yn_ref[...] = yn
        mu_ref[...] = jnp.sum(yn * yn).reshape(1, 1) / y.shape[0]


def _combine(num2, Rsel, b_row, Wn, AAn, project):
    full = lambda shape: pl.BlockSpec(shape, lambda: tuple(0 for _ in shape))
    out_shapes = [
        jax.ShapeDtypeStruct((N, DP), jnp.float32),
        jax.ShapeDtypeStruct((N, 4), jnp.float32),
        jax.ShapeDtypeStruct((N_DRUG, D), jnp.float32),
        jax.ShapeDtypeStruct((N_TARGET, D), jnp.float32),
        jax.ShapeDtypeStruct((1, 1), jnp.float32),
        jax.ShapeDtypeStruct((1, 1), jnp.float32),
    ]
    out_specs = [full((N, DP)), full((N, 4)), full((N_DRUG, D)),
                 full((N_TARGET, D)), full((1, 1)), full((1, 1))]
    res = pl.pallas_call(
        functools.partial(_combine_body, project=project),
        in_specs=[full((2, N, DP)), full((HEADS, D)),
                  full((1, D)), full((D, D)), full((D, 4))],
        out_specs=out_specs,
        out_shape=out_shapes,
    )(num2, Rsel, b_row, Wn, AAn)
    return res


def _gip_body(acc_ref, yt_ref, ynf_ref, mu_ref, o_ref, *, gamma, ps, acc_scale):
    yt = yt_ref[...]
    ynf = ynf_ref[...]
    kt = lax.dot_general(yt, ynf, (((1,), (1,)), ((), ())),
                         precision=_P, preferred_element_type=jnp.float32)
    sqni = jnp.sum(yt * yt, axis=1, keepdims=True)
    ones = jnp.ones((1, D), jnp.float32)
    sqnj = lax.dot_general(ones, ynf * ynf, (((1,), (1,)), ((), ())),
                           precision=_P, preferred_element_type=jnp.float32)
    inv_mu = 1.0 / mu_ref[0, 0]
    d = (sqni + sqnj - 2.0 * kt) * inv_mu
    o_ref[...] = acc_ref[...] * acc_scale + ps * jnp.exp(-d * gamma)


def _gip_accum(acc, yn, mu, gamma, ps, acc_scale):
    n = yn.shape[0]
    blk = 256
    return pl.pallas_call(
        functools.partial(_gip_body, gamma=gamma, ps=ps, acc_scale=acc_scale),
        grid=(n // blk,),
        in_specs=[
            pl.BlockSpec((blk, n), lambda i: (i, 0)),
            pl.BlockSpec((blk, D), lambda i: (i, 0)),
            pl.BlockSpec((n, D), lambda i: (0, 0)),
            pl.BlockSpec((1, 1), lambda i: (0, 0)),
        ],
        out_specs=pl.BlockSpec((blk, n), lambda i: (i, 0)),
        out_shape=jax.ShapeDtypeStruct((n, n), jnp.float32),
        input_output_aliases={0: 0},
    )(acc, yn, yn, mu)


def _diag_body(kd_ref, kt_ref, dd_ref, dt_ref, mn_ref, mn_acc):
    j = pl.program_id(0)
    blk = kd_ref.shape[1]
    n = kd_ref.shape[0]
    rows = lax.broadcasted_iota(jnp.int32, (n, blk), 0)
    cols = lax.broadcasted_iota(jnp.int32, (n, blk), 1)
    mask = rows == cols + j * blk

    @pl.when(j == 0)
    def _init():
        mn_acc[0] = jnp.inf
        mn_acc[1] = jnp.inf

    for k_ref, d_ref, slot in ((kd_ref, dd_ref, 0), (kt_ref, dt_ref, 1)):
        k = jnp.abs(k_ref[...])
        d_ref[...] = jnp.sum(jnp.where(mask, k, 0.0), axis=0, keepdims=True)
        pos = jnp.where(k > 0, k, jnp.inf)
        mn_acc[slot] = jnp.minimum(mn_acc[slot], jnp.min(pos))
    mn_ref[0, 0] = mn_acc[0]
    mn_ref[0, 1] = mn_acc[1]


def _diag_minpos(Kd, Kt):
    blk = 256
    return pl.pallas_call(
        _diag_body,
        grid=(N_DRUG // blk,),
        in_specs=[
            pl.BlockSpec((N_DRUG, blk), lambda j: (0, j)),
            pl.BlockSpec((N_TARGET, blk), lambda j: (0, j)),
        ],
        out_specs=[
            pl.BlockSpec((1, blk), lambda j: (0, j)),
            pl.BlockSpec((1, blk), lambda j: (0, j)),
            pl.BlockSpec((1, 2), lambda j: (0, 0), memory_space=pltpu.SMEM),
        ],
        out_shape=[
            jax.ShapeDtypeStruct((1, N_DRUG), jnp.float32),
            jax.ShapeDtypeStruct((1, N_TARGET), jnp.float32),
            jax.ShapeDtypeStruct((1, 2), jnp.float32),
        ],
        scratch_shapes=[pltpu.SMEM((2,), jnp.float32)],
    )(Kd, Kt)


def _final_body(kd_ref, kt_ref, a1_ref, a2_ref, dd_ref, dt_ref, mn_ref, o_ref):
    mnd = mn_ref[0, 0]
    mnt = mn_ref[0, 1]
    dd = dd_ref[...]
    dt = dt_ref[...]
    sd = 1.0 / jnp.where(dd == 0, mnd, dd)
    st = 1.0 / jnp.where(dt == 0, mnt, dt)
    kd = jnp.abs(kd_ref[...])
    kd = jnp.where(kd == 0, mnd, kd) * sd
    kt = jnp.abs(kt_ref[...])
    kt = jnp.where(kt == 0, mnt, kt) * st
    t1 = _dot(kd, a1_ref[...])
    t2 = lax.dot_general(a2_ref[...], kt, (((0,), (1,)), ((), ())),
                         precision=_P, preferred_element_type=jnp.float32)
    o_ref[...] = 0.5 * (t1 + t2)


def _final(Kd, Kt, a1, a2, dd, dt, mn):
    blk = 256
    return pl.pallas_call(
        _final_body,
        grid=(N_DRUG // blk,),
        in_specs=[
            pl.BlockSpec((blk, N_DRUG), lambda i: (i, 0)),
            pl.BlockSpec((N_TARGET, N_TARGET), lambda i: (0, 0)),
            pl.BlockSpec((N_DRUG, N_TARGET), lambda i: (0, 0)),
            pl.BlockSpec((N_TARGET, blk), lambda i: (0, i)),
            pl.BlockSpec((1, N_DRUG), lambda i: (0, 0)),
            pl.BlockSpec((1, N_TARGET), lambda i: (0, 0)),
            pl.BlockSpec((1, 2), lambda i: (0, 0), memory_space=pltpu.SMEM),
        ],
        out_specs=pl.BlockSpec((blk, N_TARGET), lambda i: (i, 0)),
        out_shape=jax.ShapeDtypeStruct((N_DRUG, N_TARGET), jnp.float32),
    )(Kd, Kt, a1, a2, dd, dt, mn)


NWORK = 32
EW = EFULL // NWORK
CH = 96
NCHUNK = EW // CH
NGROUP = CH // 16


def _edge_body(h_hbm, sc_hbm, src_hbm, dst_hbm, dst2_hbm, z2_hbm,
               num_hbm,
               sc_tab, src1_v, dst1_v, dst2_v, ex0_v, ex1_v,
               rows0_v, rows1_v,
               num_sp, sg0, sg1, ss0, ss1):
    c = lax.axis_index("c")
    s = lax.axis_index("s")
    wid = s * 2 + c
    base = pl.multiple_of(wid * EW, 8)
    rows = (rows0_v, rows1_v)
    sg = (sg0, sg1)
    ss = (ss0, ss1)

    pltpu.sync_copy(sc_hbm, sc_tab)
    pltpu.sync_copy(src_hbm.at[pl.ds(base, EW)], src1_v)
    pltpu.sync_copy(dst_hbm.at[pl.ds(base, EW)], dst1_v)
    pltpu.sync_copy(dst2_hbm.at[wid], dst2_v)

    nsl = N // 16
    pltpu.sync_copy(z2_hbm.at[pl.ds(s * nsl, nsl)],
                    num_sp.at[pl.ds(s * nsl, nsl)])

    d_g0 = pltpu.async_copy(h_hbm.at[src1_v.at[pl.ds(0, CH)]], rows0_v, sg0)

    def group_body(g, carry2):
        go = g * 16
        sidx = src1_v[pl.ds(go, 16)]
        didx = dst1_v[pl.ds(go, 16)]
        as0 = plsc.load_gather(sc_tab, [sidx * 4])
        as1 = plsc.load_gather(sc_tab, [sidx * 4 + 1])
        ad0 = plsc.load_gather(sc_tab, [didx * 4 + 2])
        ad1 = plsc.load_gather(sc_tab, [didx * 4 + 3])
        e0 = as0 + ad0
        e1 = as1 + ad1
        e0 = jnp.where(e0 > 0, e0, 0.2 * e0)
        e1 = jnp.where(e1 > 0, e1, 0.2 * e1)
        ex0_v[pl.ds(go, 16)] = jnp.exp(e0)
        ex1_v[pl.ds(go, 16)] = jnp.exp(e1)
        return carry2

    lax.fori_loop(0, EW // 16, group_body, 0)
    plsc.subcore_barrier()
    iot = lax.iota(jnp.int32, 16)
    colD = jnp.full((16,), D, jnp.int32)

    d_s = [None] * NCHUNK
    d_g = [None] * NCHUNK
    d_g[0] = d_g0
    for k in range(NCHUNK):
        p = k & 1
        q = 1 - p
        rp, rq = rows[p], rows[q]
        if k + 1 < NCHUNK:
            if k >= 1:
                d_s[k - 1].wait()
            d_g[k + 1] = pltpu.async_copy(
                h_hbm.at[src1_v.at[pl.ds((k + 1) * CH, CH)]], rq, sg[q])
        d_g[k].wait()

        def den_body(g, carry2, rp=rp, k=k):
            go = g * 16
            eo = k * CH + go
            plsc.store_scatter(rp, [go + iot, colD], ex0_v[pl.ds(eo, 16)])
            plsc.store_scatter(rp, [go + iot, colD + 1],
                               ex1_v[pl.ds(eo, 16)])
            return carry2

        lax.fori_loop(0, NGROUP, den_body, 0)

        @plsc.parallel_loop(0, CH, step=1, unroll=4)
        def edge_body(e, rp=rp, k=k):
            i0 = jnp.full((16,), k * CH + e, jnp.int32)
            w0 = plsc.load_gather(ex0_v, [i0])
            w1 = plsc.load_gather(ex1_v, [i0])
            rp[e, pl.ds(0, 16)] = rp[e, pl.ds(0, 16)] * w0
            rp[e, pl.ds(16, 16)] = rp[e, pl.ds(16, 16)] * w0
            rp[e, pl.ds(32, 16)] = rp[e, pl.ds(32, 16)] * w1
            rp[e, pl.ds(48, 16)] = rp[e, pl.ds(48, 16)] * w1

        d_s[k] = pltpu.async_copy(rp, num_sp.at[dst2_v.at[k]], ss[p],
                                  add=True)

    d_s[NCHUNK - 2].wait()
    d_s[NCHUNK - 1].wait()
    plsc.subcore_barrier()

    pltpu.sync_copy(num_sp.at[pl.ds(s * nsl, nsl)],
                    num_hbm.at[c, pl.ds(s * nsl, nsl)])


def _edges_sc(h, scflat, srcfull, dstfull, dst2):
    mesh = plsc.VectorSubcoreMesh(core_axis_name="c", subcore_axis_name="s")
    z2 = jnp.zeros((N, DP), jnp.float32)
    call = pl.kernel(
        _edge_body,
        out_type=jax.ShapeDtypeStruct((2, N, DP), jnp.float32),
        mesh=mesh,
        compiler_params=pltpu.CompilerParams(needs_layout_passes=False),
        scratch_types=[
            pltpu.VMEM((4 * N,), jnp.float32),
            pltpu.VMEM((EW,), jnp.int32),
            pltpu.VMEM((EW,), jnp.int32),
            pltpu.VMEM((NCHUNK, CH), jnp.int32),
            pltpu.VMEM((EW,), jnp.float32),
            pltpu.VMEM((EW,), jnp.float32),
            pltpu.VMEM((CH, DP), jnp.float32),
            pltpu.VMEM((CH, DP), jnp.float32),
            pltpu.VMEM_SHARED((N, DP), jnp.float32),
            pltpu.SemaphoreType.DMA,
            pltpu.SemaphoreType.DMA,
            pltpu.SemaphoreType.DMA,
            pltpu.SemaphoreType.DMA,
        ],
    )
    return call(h, scflat, srcfull, dstfull, dst2, z2)


def kernel(x, Adj_data, edge_index, W1, a_src1, a_dst1, b1, W2, a_src2,
           a_dst2, b2, W3, a_src3, a_dst3, b3, W4, a_src4, a_dst4, b4,
           alpha1, alpha2, drug_sim, target_sim):
    del Adj_data

    loop = jnp.arange(N, dtype=edge_index.dtype)
    srcfull = jnp.concatenate([edge_index[0], loop])
    dstfull = jnp.concatenate([edge_index[1], loop])
    dst2 = dstfull.reshape(NWORK, NCHUNK, CH)

    Rsel = jnp.repeat(jnp.eye(HEADS, dtype=jnp.float32), F, axis=1)

    def make_aa(a_src, a_dst):
        z = jnp.zeros((F,), jnp.float32)
        c0 = jnp.concatenate([a_src[0], z])
        c1 = jnp.concatenate([z, a_src[1]])
        c2 = jnp.concatenate([a_dst[0], z])
        c3 = jnp.concatenate([z, a_dst[1]])
        return jnp.stack([c0, c1, c2, c3], axis=1)

    AAs = [make_aa(a_src1, a_dst1), make_aa(a_src2, a_dst2),
           make_aa(a_src3, a_dst3), make_aa(a_src4, a_dst4)]
    Ws = [W1, W2, W3, W4]
    bs = [b1.reshape(1, D), b2.reshape(1, D), b3.reshape(1, D),
          b4.reshape(1, D)]

    drug_ps = (1.0, 0.5, 0.333, 0.25)
    target_ps = (0.2, 0.2, 0.2, 0.2)

    h, sc = _proj1(x, W1, AAs[0])

    kd_acc = drug_sim
    kt_acc = target_sim
    dummyW = jnp.zeros((D, D), jnp.float32)
    dummyAA = jnp.zeros((D, 4), jnp.float32)
    for layer in range(4):
        num2 = _edges_sc(h, sc.reshape(-1), srcfull, dstfull, dst2)
        last = layer == 3
        Wn = Ws[layer + 1] if not last else dummyW
        AAn = AAs[layer + 1] if not last else dummyAA
        h, sc, ynd, ynt, mud, mut = _combine(
            num2, Rsel, bs[layer], Wn, AAn, project=not last)
        acc_scale = 0.2 if layer == 0 else 1.0
        kd_acc = _gip_accum(kd_acc, ynd, mud, GAMMAS[layer],
                            drug_ps[layer], acc_scale)
        kt_acc = _gip_accum(kt_acc, ynt, mut, GAMMAS[layer],
                            target_ps[layer], acc_scale)

    dd, dt, mn = _diag_minpos(kd_acc, kt_acc)
    return _final(kd_acc, kt_acc, alpha1, alpha2, dd, dt, mn)


# speedup vs baseline: 108.9774x; 1.0098x over previous
import functools

import jax
import jax.numpy as jnp
from jax import lax
from jax.experimental import pallas as pl
from jax.experimental.pallas import tpu as pltpu
from jax.experimental.pallas import tpu_sc as plsc

N_DRUG = 1024
N_TARGET = 1024
N = N_DRUG + N_TARGET
E = 65536
EFULL = E + N
HEADS = 2
F = 32
D = HEADS * F
DP = 128
GAMMAS = (0.01, 0.01, 0.01, 0.01)

_P = jax.lax.Precision.HIGHEST


def _dot(a, b, dims=(((1,), (0,)), ((), ()))):
    return lax.dot_general(a, b, dims, precision=_P,
                           preferred_element_type=jnp.float32)


def _proj1_body(x_ref, w_ref, aa_ref, h_ref, sc_ref):
    h = _dot(x_ref[...], w_ref[...])
    h_ref[...] = jnp.concatenate(
        [h, jnp.zeros((h.shape[0], DP - D), jnp.float32)], axis=1)
    sc_ref[...] = _dot(h, aa_ref[...])


def _proj1(x, W1, AA):
    blk = 256
    return pl.pallas_call(
        _proj1_body,
        grid=(N // blk,),
        in_specs=[
            pl.BlockSpec((blk, N), lambda i: (i, 0)),
            pl.BlockSpec((N, D), lambda i: (0, 0)),
            pl.BlockSpec((D, 4), lambda i: (0, 0)),
        ],
        out_specs=[
            pl.BlockSpec((blk, DP), lambda i: (i, 0)),
            pl.BlockSpec((blk, 4), lambda i: (i, 0)),
        ],
        out_shape=[
            jax.ShapeDtypeStruct((N, DP), jnp.float32),
            jax.ShapeDtypeStruct((N, 4), jnp.float32),
        ],
    )(x, W1, AA)


def _combine_body(num_ref, r_ref, b_ref, w_ref, aa_ref,
                  h_ref, sc_ref, ynd_ref, ynt_ref, mud_ref, mut_ref,
                  *, project):
    acc = num_ref[0] + num_ref[1]
    num = acc[:, :D]
    den = acc[:, D:D + HEADS]
    recip = 1.0 / (den + 1e-16)
    scale = _dot(recip, r_ref[...])
    H = jnp.maximum(num * scale + b_ref[...], 0.0)
    pad = jnp.zeros((N, DP - D), jnp.float32)
    if project:
        h = _dot(H, w_ref[...])
        h_ref[...] = jnp.concatenate([h, pad], axis=1)
        sc_ref[...] = _dot(h, aa_ref[...])
    else:
        h_ref[...] = jnp.concatenate([H, pad], axis=1)
        sc_ref[...] = jnp.zeros_like(sc_ref)
    yd = H[:N_DRUG]
    yt = H[N_DRUG:]
    for y, yn_ref, mu_ref in ((yd, ynd_ref, mud_ref), (yt, ynt_ref, mut_ref)):
        mn = jnp.min(y, axis=1, keepdims=True)
        mx = jnp.max(y, axis=1, keepdims=True)
        yn = (y - mn) / (mx - mn + 1e-12)
        yn_ref[...] = yn
        mu_ref[...] = jnp.sum(yn * yn).reshape(1, 1) / y.shape[0]


def _combine(num2, Rsel, b_row, Wn, AAn, project):
    full = lambda shape: pl.BlockSpec(shape, lambda: tuple(0 for _ in shape))
    out_shapes = [
        jax.ShapeDtypeStruct((N, DP), jnp.float32),
        jax.ShapeDtypeStruct((N, 4), jnp.float32),
        jax.ShapeDtypeStruct((N_DRUG, D), jnp.float32),
        jax.ShapeDtypeStruct((N_TARGET, D), jnp.float32),
        jax.ShapeDtypeStruct((1, 1), jnp.float32),
        jax.ShapeDtypeStruct((1, 1), jnp.float32),
    ]
    out_specs = [full((N, DP)), full((N, 4)), full((N_DRUG, D)),
                 full((N_TARGET, D)), full((1, 1)), full((1, 1))]
    res = pl.pallas_call(
        functools.partial(_combine_body, project=project),
        in_specs=[full((2, N, DP)), full((HEADS, D)),
                  full((1, D)), full((D, D)), full((D, 4))],
        out_specs=out_specs,
        out_shape=out_shapes,
    )(num2, Rsel, b_row, Wn, AAn)
    return res


def _gip_half(acc, yt, ynf, mu, gamma, ps, acc_scale):
    kt = lax.dot_general(yt, ynf, (((1,), (1,)), ((), ())),
                         precision=_P, preferred_element_type=jnp.float32)
    sqni = jnp.sum(yt * yt, axis=1, keepdims=True)
    ones = jnp.ones((1, D), jnp.float32)
    sqnj = lax.dot_general(ones, ynf * ynf, (((1,), (1,)), ((), ())),
                           precision=_P, preferred_element_type=jnp.float32)
    d = (sqni + sqnj - 2.0 * kt) / mu
    return acc * acc_scale + ps * jnp.exp(-d * gamma)


def _gip_body(accd_ref, acct_ref, ytd_ref, ynd_ref, ytt_ref, ynt_ref,
              mud_ref, mut_ref, od_ref, ot_ref, *, gamma, psd, pst,
              acc_scale):
    od_ref[...] = _gip_half(accd_ref[...], ytd_ref[...], ynd_ref[...],
                            mud_ref[0, 0], gamma, psd, acc_scale)
    ot_ref[...] = _gip_half(acct_ref[...], ytt_ref[...], ynt_ref[...],
                            mut_ref[0, 0], gamma, pst, acc_scale)


def _gip_accum(accd, acct, ynd, ynt, mud, mut, gamma, psd, pst, acc_scale):
    n = N_DRUG
    blk = 256
    return pl.pallas_call(
        functools.partial(_gip_body, gamma=gamma, psd=psd, pst=pst,
                          acc_scale=acc_scale),
        grid=(n // blk,),
        in_specs=[
            pl.BlockSpec((blk, n), lambda i: (i, 0)),
            pl.BlockSpec((blk, n), lambda i: (i, 0)),
            pl.BlockSpec((blk, D), lambda i: (i, 0)),
            pl.BlockSpec((n, D), lambda i: (0, 0)),
            pl.BlockSpec((blk, D), lambda i: (i, 0)),
            pl.BlockSpec((n, D), lambda i: (0, 0)),
            pl.BlockSpec((1, 1), lambda i: (0, 0)),
            pl.BlockSpec((1, 1), lambda i: (0, 0)),
        ],
        out_specs=[
            pl.BlockSpec((blk, n), lambda i: (i, 0)),
            pl.BlockSpec((blk, n), lambda i: (i, 0)),
        ],
        out_shape=[
            jax.ShapeDtypeStruct((n, n), jnp.float32),
            jax.ShapeDtypeStruct((n, n), jnp.float32),
        ],
        input_output_aliases={0: 0, 1: 1},
    )(accd, acct, ynd, ynd, ynt, ynt, mud, mut)


def _diag_body(kd_ref, kt_ref, dd_ref, dt_ref, mn_ref, mn_acc):
    j = pl.program_id(0)
    blk = kd_ref.shape[1]
    n = kd_ref.shape[0]
    rows = lax.broadcasted_iota(jnp.int32, (n, blk), 0)
    cols = lax.broadcasted_iota(jnp.int32, (n, blk), 1)
    mask = rows == cols + j * blk

    @pl.when(j == 0)
    def _init():
        mn_acc[0] = jnp.inf
        mn_acc[1] = jnp.inf

    for k_ref, d_ref, slot in ((kd_ref, dd_ref, 0), (kt_ref, dt_ref, 1)):
        k = jnp.abs(k_ref[...])
        d_ref[...] = jnp.sum(jnp.where(mask, k, 0.0), axis=0, keepdims=True)
        pos = jnp.where(k > 0, k, jnp.inf)
        mn_acc[slot] = jnp.minimum(mn_acc[slot], jnp.min(pos))
    mn_ref[0, 0] = mn_acc[0]
    mn_ref[0, 1] = mn_acc[1]


def _diag_minpos(Kd, Kt):
    blk = 256
    return pl.pallas_call(
        _diag_body,
        grid=(N_DRUG // blk,),
        in_specs=[
            pl.BlockSpec((N_DRUG, blk), lambda j: (0, j)),
            pl.BlockSpec((N_TARGET, blk), lambda j: (0, j)),
        ],
        out_specs=[
            pl.BlockSpec((1, blk), lambda j: (0, j)),
            pl.BlockSpec((1, blk), lambda j: (0, j)),
            pl.BlockSpec((1, 2), lambda j: (0, 0), memory_space=pltpu.SMEM),
        ],
        out_shape=[
            jax.ShapeDtypeStruct((1, N_DRUG), jnp.float32),
            jax.ShapeDtypeStruct((1, N_TARGET), jnp.float32),
            jax.ShapeDtypeStruct((1, 2), jnp.float32),
        ],
        scratch_shapes=[pltpu.SMEM((2,), jnp.float32)],
    )(Kd, Kt)


def _final_body(kd_ref, kt_ref, a1_ref, a2_ref, dd_ref, dt_ref, mn_ref, o_ref):
    mnd = mn_ref[0, 0]
    mnt = mn_ref[0, 1]
    dd = dd_ref[...]
    dt = dt_ref[...]
    sd = 1.0 / jnp.where(dd == 0, mnd, dd)
    st = 1.0 / jnp.where(dt == 0, mnt, dt)
    kd = jnp.abs(kd_ref[...])
    kd = jnp.where(kd == 0, mnd, kd) * sd
    kt = jnp.abs(kt_ref[...])
    kt = jnp.where(kt == 0, mnt, kt) * st
    t1 = _dot(kd, a1_ref[...])
    t2 = lax.dot_general(a2_ref[...], kt, (((0,), (1,)), ((), ())),
                         precision=_P, preferred_element_type=jnp.float32)
    o_ref[...] = 0.5 * (t1 + t2)


def _final(Kd, Kt, a1, a2, dd, dt, mn):
    blk = 256
    return pl.pallas_call(
        _final_body,
        grid=(N_DRUG // blk,),
        in_specs=[
            pl.BlockSpec((blk, N_DRUG), lambda i: (i, 0)),
            pl.BlockSpec((N_TARGET, N_TARGET), lambda i: (0, 0)),
            pl.BlockSpec((N_DRUG, N_TARGET), lambda i: (0, 0)),
            pl.BlockSpec((N_TARGET, blk), lambda i: (0, i)),
            pl.BlockSpec((1, N_DRUG), lambda i: (0, 0)),
            pl.BlockSpec((1, N_TARGET), lambda i: (0, 0)),
            pl.BlockSpec((1, 2), lambda i: (0, 0), memory_space=pltpu.SMEM),
        ],
        out_specs=pl.BlockSpec((blk, N_TARGET), lambda i: (i, 0)),
        out_shape=jax.ShapeDtypeStruct((N_DRUG, N_TARGET), jnp.float32),
    )(Kd, Kt, a1, a2, dd, dt, mn)


NWORK = 32
EW = EFULL // NWORK
CH = 96
NCHUNK = EW // CH
NGROUP = CH // 16


def _edge_body(h_hbm, sc_hbm, src_hbm, dst_hbm, dst2_hbm, z2_hbm,
               num_hbm,
               sc_tab, src1_v, dst1_v, dst2_v, ex0_v, ex1_v,
               rows0_v, rows1_v,
               num_sp, sg0, sg1, ss0, ss1):
    c = lax.axis_index("c")
    s = lax.axis_index("s")
    wid = s * 2 + c
    base = pl.multiple_of(wid * EW, 8)
    rows = (rows0_v, rows1_v)
    sg = (sg0, sg1)
    ss = (ss0, ss1)

    pltpu.sync_copy(sc_hbm, sc_tab)
    pltpu.sync_copy(src_hbm.at[pl.ds(base, EW)], src1_v)
    pltpu.sync_copy(dst_hbm.at[pl.ds(base, EW)], dst1_v)
    pltpu.sync_copy(dst2_hbm.at[wid], dst2_v)

    nsl = N // 16
    pltpu.sync_copy(z2_hbm.at[pl.ds(s * nsl, nsl)],
                    num_sp.at[pl.ds(s * nsl, nsl)])

    d_g0 = pltpu.async_copy(h_hbm.at[src1_v.at[pl.ds(0, CH)]], rows0_v, sg0)

    def group_body(g, carry2):
        go = g * 16
        sidx = src1_v[pl.ds(go, 16)]
        didx = dst1_v[pl.ds(go, 16)]
        as0 = plsc.load_gather(sc_tab, [sidx * 4])
        as1 = plsc.load_gather(sc_tab, [sidx * 4 + 1])
        ad0 = plsc.load_gather(sc_tab, [didx * 4 + 2])
        ad1 = plsc.load_gather(sc_tab, [didx * 4 + 3])
        e0 = as0 + ad0
        e1 = as1 + ad1
        e0 = jnp.where(e0 > 0, e0, 0.2 * e0)
        e1 = jnp.where(e1 > 0, e1, 0.2 * e1)
        ex0_v[pl.ds(go, 16)] = jnp.exp(e0)
        ex1_v[pl.ds(go, 16)] = jnp.exp(e1)
        return carry2

    lax.fori_loop(0, EW // 16, group_body, 0)
    plsc.subcore_barrier()
    iot = lax.iota(jnp.int32, 16)
    colD = jnp.full((16,), D, jnp.int32)

    d_s = [None] * NCHUNK
    d_g = [None] * NCHUNK
    d_g[0] = d_g0
    for k in range(NCHUNK):
        p = k & 1
        q = 1 - p
        rp, rq = rows[p], rows[q]
        if k + 1 < NCHUNK:
            if k >= 1:
                d_s[k - 1].wait()
            d_g[k + 1] = pltpu.async_copy(
                h_hbm.at[src1_v.at[pl.ds((k + 1) * CH, CH)]], rq, sg[q])
        d_g[k].wait()

        def den_body(g, carry2, rp=rp, k=k):
            go = g * 16
            eo = k * CH + go
            plsc.store_scatter(rp, [go + iot, colD], ex0_v[pl.ds(eo, 16)])
            plsc.store_scatter(rp, [go + iot, colD + 1],
                               ex1_v[pl.ds(eo, 16)])
            return carry2

        lax.fori_loop(0, NGROUP, den_body, 0)

        @plsc.parallel_loop(0, CH, step=1, unroll=8)
        def edge_body(e, rp=rp, k=k):
            i0 = jnp.full((16,), k * CH + e, jnp.int32)
            w0 = plsc.load_gather(ex0_v, [i0])
            w1 = plsc.load_gather(ex1_v, [i0])
            rp[e, pl.ds(0, 16)] = rp[e, pl.ds(0, 16)] * w0
            rp[e, pl.ds(16, 16)] = rp[e, pl.ds(16, 16)] * w0
            rp[e, pl.ds(32, 16)] = rp[e, pl.ds(32, 16)] * w1
            rp[e, pl.ds(48, 16)] = rp[e, pl.ds(48, 16)] * w1

        d_s[k] = pltpu.async_copy(rp, num_sp.at[dst2_v.at[k]], ss[p],
                                  add=True)

    d_s[NCHUNK - 2].wait()
    d_s[NCHUNK - 1].wait()
    plsc.subcore_barrier()

    pltpu.sync_copy(num_sp.at[pl.ds(s * nsl, nsl)],
                    num_hbm.at[c, pl.ds(s * nsl, nsl)])


def _edges_sc(h, scflat, srcfull, dstfull, dst2):
    mesh = plsc.VectorSubcoreMesh(core_axis_name="c", subcore_axis_name="s")
    z2 = jnp.zeros((N, DP), jnp.float32)
    call = pl.kernel(
        _edge_body,
        out_type=jax.ShapeDtypeStruct((2, N, DP), jnp.float32),
        mesh=mesh,
        compiler_params=pltpu.CompilerParams(needs_layout_passes=False),
        scratch_types=[
            pltpu.VMEM((4 * N,), jnp.float32),
            pltpu.VMEM((EW,), jnp.int32),
            pltpu.VMEM((EW,), jnp.int32),
            pltpu.VMEM((NCHUNK, CH), jnp.int32),
            pltpu.VMEM((EW,), jnp.float32),
            pltpu.VMEM((EW,), jnp.float32),
            pltpu.VMEM((CH, DP), jnp.float32),
            pltpu.VMEM((CH, DP), jnp.float32),
            pltpu.VMEM_SHARED((N, DP), jnp.float32),
            pltpu.SemaphoreType.DMA,
            pltpu.SemaphoreType.DMA,
            pltpu.SemaphoreType.DMA,
            pltpu.SemaphoreType.DMA,
        ],
    )
    return call(h, scflat, srcfull, dstfull, dst2, z2)


def kernel(x, Adj_data, edge_index, W1, a_src1, a_dst1, b1, W2, a_src2,
           a_dst2, b2, W3, a_src3, a_dst3, b3, W4, a_src4, a_dst4, b4,
           alpha1, alpha2, drug_sim, target_sim):
    del Adj_data

    loop = jnp.arange(N, dtype=edge_index.dtype)
    srcfull = jnp.concatenate([edge_index[0], loop])
    dstfull = jnp.concatenate([edge_index[1], loop])
    dst2 = dstfull.reshape(NWORK, NCHUNK, CH)

    Rsel = jnp.repeat(jnp.eye(HEADS, dtype=jnp.float32), F, axis=1)

    def make_aa(a_src, a_dst):
        z = jnp.zeros((F,), jnp.float32)
        c0 = jnp.concatenate([a_src[0], z])
        c1 = jnp.concatenate([z, a_src[1]])
        c2 = jnp.concatenate([a_dst[0], z])
        c3 = jnp.concatenate([z, a_dst[1]])
        return jnp.stack([c0, c1, c2, c3], axis=1)

    AAs = [make_aa(a_src1, a_dst1), make_aa(a_src2, a_dst2),
           make_aa(a_src3, a_dst3), make_aa(a_src4, a_dst4)]
    Ws = [W1, W2, W3, W4]
    bs = [b1.reshape(1, D), b2.reshape(1, D), b3.reshape(1, D),
          b4.reshape(1, D)]

    drug_ps = (1.0, 0.5, 0.333, 0.25)
    target_ps = (0.2, 0.2, 0.2, 0.2)

    h, sc = _proj1(x, W1, AAs[0])

    kd_acc = drug_sim
    kt_acc = target_sim
    dummyW = jnp.zeros((D, D), jnp.float32)
    dummyAA = jnp.zeros((D, 4), jnp.float32)
    for layer in range(4):
        num2 = _edges_sc(h, sc.reshape(-1), srcfull, dstfull, dst2)
        last = layer == 3
        Wn = Ws[layer + 1] if not last else dummyW
        AAn = AAs[layer + 1] if not last else dummyAA
        h, sc, ynd, ynt, mud, mut = _combine(
            num2, Rsel, bs[layer], Wn, AAn, project=not last)
        acc_scale = 0.2 if layer == 0 else 1.0
        kd_acc, kt_acc = _gip_accum(kd_acc, kt_acc, ynd, ynt, mud, mut,
                                    GAMMAS[layer], drug_ps[layer],
                                    target_ps[layer], acc_scale)

    dd, dt, mn = _diag_minpos(kd_acc, kt_acc)
    return _final(kd_acc, kt_acc, alpha1, alpha2, dd, dt, mn)


# speedup vs baseline: 128.5590x; 1.1797x over previous
import functools

import jax
import jax.numpy as jnp
from jax import lax
from jax.experimental import pallas as pl
from jax.experimental.pallas import tpu as pltpu
from jax.experimental.pallas import tpu_sc as plsc

N_DRUG = 1024
N_TARGET = 1024
N = N_DRUG + N_TARGET
E = 65536
EFULL = E + N
HEADS = 2
F = 32
D = HEADS * F
DP = 128
GAMMAS = (0.01, 0.01, 0.01, 0.01)

_P = jax.lax.Precision.HIGHEST


def _dot(a, b, dims=(((1,), (0,)), ((), ())), prec=_P):
    return lax.dot_general(a, b, dims, precision=prec,
                           preferred_element_type=jnp.float32)


def _proj1_body(x_ref, w_ref, aa_ref, h_ref, sc_ref):
    h = _dot(x_ref[...], w_ref[...], prec=jax.lax.Precision.DEFAULT)
    h_ref[...] = jnp.concatenate(
        [h, jnp.zeros((h.shape[0], DP - D), jnp.float32)], axis=1)
    sc_ref[...] = _dot(h, aa_ref[...])


def _proj1(x, W1, AA):
    blk = 256
    return pl.pallas_call(
        _proj1_body,
        grid=(N // blk,),
        in_specs=[
            pl.BlockSpec((blk, N), lambda i: (i, 0)),
            pl.BlockSpec((N, D), lambda i: (0, 0)),
            pl.BlockSpec((D, 4), lambda i: (0, 0)),
        ],
        out_specs=[
            pl.BlockSpec((blk, DP), lambda i: (i, 0)),
            pl.BlockSpec((blk, 4), lambda i: (i, 0)),
        ],
        out_shape=[
            jax.ShapeDtypeStruct((N, DP), jnp.float32),
            jax.ShapeDtypeStruct((N, 4), jnp.float32),
        ],
    )(x, W1, AA)


def _combine_body(num_ref, r_ref, b_ref, w_ref, aa_ref,
                  h_ref, sc_ref, ynd_ref, ynt_ref, mud_ref, mut_ref,
                  *, project):
    acc = num_ref[0] + num_ref[1]
    num = acc[:, :D]
    den = acc[:, D:D + HEADS]
    recip = 1.0 / (den + 1e-16)
    scale = _dot(recip, r_ref[...])
    H = jnp.maximum(num * scale + b_ref[...], 0.0)
    pad = jnp.zeros((N, DP - D), jnp.float32)
    if project:
        h = _dot(H, w_ref[...])
        h_ref[...] = jnp.concatenate([h, pad], axis=1)
        sc_ref[...] = _dot(h, aa_ref[...])
    else:
        h_ref[...] = jnp.concatenate([H, pad], axis=1)
        sc_ref[...] = jnp.zeros_like(sc_ref)
    yd = H[:N_DRUG]
    yt = H[N_DRUG:]
    for y, yn_ref, mu_ref in ((yd, ynd_ref, mud_ref), (yt, ynt_ref, mut_ref)):
        mn = jnp.min(y, axis=1, keepdims=True)
        mx = jnp.max(y, axis=1, keepdims=True)
        yn = (y - mn) / (mx - mn + 1e-12)
        yn_ref[...] = yn
        mu_ref[...] = jnp.sum(yn * yn).reshape(1, 1) / y.shape[0]


def _combine(num2, Rsel, b_row, Wn, AAn, project):
    full = lambda shape: pl.BlockSpec(shape, lambda: tuple(0 for _ in shape))
    out_shapes = [
        jax.ShapeDtypeStruct((N, DP), jnp.float32),
        jax.ShapeDtypeStruct((N, 4), jnp.float32),
        jax.ShapeDtypeStruct((N_DRUG, D), jnp.float32),
        jax.ShapeDtypeStruct((N_TARGET, D), jnp.float32),
        jax.ShapeDtypeStruct((1, 1), jnp.float32),
        jax.ShapeDtypeStruct((1, 1), jnp.float32),
    ]
    out_specs = [full((N, DP)), full((N, 4)), full((N_DRUG, D)),
                 full((N_TARGET, D)), full((1, 1)), full((1, 1))]
    res = pl.pallas_call(
        functools.partial(_combine_body, project=project),
        in_specs=[full((2, N, DP)), full((HEADS, D)),
                  full((1, D)), full((D, D)), full((D, 4))],
        out_specs=out_specs,
        out_shape=out_shapes,
    )(num2, Rsel, b_row, Wn, AAn)
    return res


def _gip_half(acc, yt, ynf, mu, gamma, ps, acc_scale):
    kt = lax.dot_general(yt, ynf, (((1,), (1,)), ((), ())),
                         precision=jax.lax.Precision.DEFAULT,
                         preferred_element_type=jnp.float32)
    sqni = jnp.sum(yt * yt, axis=1, keepdims=True)
    ones = jnp.ones((1, D), jnp.float32)
    sqnj = lax.dot_general(ones, ynf * ynf, (((1,), (1,)), ((), ())),
                           precision=_P, preferred_element_type=jnp.float32)
    d = (sqni + sqnj - 2.0 * kt) / mu
    return acc * acc_scale + ps * jnp.exp(-d * gamma)


def _gip_body(accd_ref, acct_ref, ytd_ref, ynd_ref, ytt_ref, ynt_ref,
              mud_ref, mut_ref, od_ref, ot_ref, *, gamma, psd, pst,
              acc_scale):
    od_ref[...] = _gip_half(accd_ref[...], ytd_ref[...], ynd_ref[...],
                            mud_ref[0, 0], gamma, psd, acc_scale)
    ot_ref[...] = _gip_half(acct_ref[...], ytt_ref[...], ynt_ref[...],
                            mut_ref[0, 0], gamma, pst, acc_scale)


def _gip_accum(accd, acct, ynd, ynt, mud, mut, gamma, psd, pst, acc_scale):
    n = N_DRUG
    blk = 256
    return pl.pallas_call(
        functools.partial(_gip_body, gamma=gamma, psd=psd, pst=pst,
                          acc_scale=acc_scale),
        grid=(n // blk,),
        in_specs=[
            pl.BlockSpec((blk, n), lambda i: (i, 0)),
            pl.BlockSpec((blk, n), lambda i: (i, 0)),
            pl.BlockSpec((blk, D), lambda i: (i, 0)),
            pl.BlockSpec((n, D), lambda i: (0, 0)),
            pl.BlockSpec((blk, D), lambda i: (i, 0)),
            pl.BlockSpec((n, D), lambda i: (0, 0)),
            pl.BlockSpec((1, 1), lambda i: (0, 0)),
            pl.BlockSpec((1, 1), lambda i: (0, 0)),
        ],
        out_specs=[
            pl.BlockSpec((blk, n), lambda i: (i, 0)),
            pl.BlockSpec((blk, n), lambda i: (i, 0)),
        ],
        out_shape=[
            jax.ShapeDtypeStruct((n, n), jnp.float32),
            jax.ShapeDtypeStruct((n, n), jnp.float32),
        ],
        input_output_aliases={0: 0, 1: 1},
    )(accd, acct, ynd, ynd, ynt, ynt, mud, mut)


def _diag_body(kd_ref, kt_ref, dd_ref, dt_ref, mn_ref, mn_acc):
    j = pl.program_id(0)
    blk = kd_ref.shape[1]
    n = kd_ref.shape[0]
    rows = lax.broadcasted_iota(jnp.int32, (n, blk), 0)
    cols = lax.broadcasted_iota(jnp.int32, (n, blk), 1)
    mask = rows == cols + j * blk

    @pl.when(j == 0)
    def _init():
        mn_acc[0] = jnp.inf
        mn_acc[1] = jnp.inf

    for k_ref, d_ref, slot in ((kd_ref, dd_ref, 0), (kt_ref, dt_ref, 1)):
        k = jnp.abs(k_ref[...])
        d_ref[...] = jnp.sum(jnp.where(mask, k, 0.0), axis=0, keepdims=True)
        pos = jnp.where(k > 0, k, jnp.inf)
        mn_acc[slot] = jnp.minimum(mn_acc[slot], jnp.min(pos))
    mn_ref[0, 0] = mn_acc[0]
    mn_ref[0, 1] = mn_acc[1]


def _diag_minpos(Kd, Kt):
    blk = 256
    return pl.pallas_call(
        _diag_body,
        grid=(N_DRUG // blk,),
        in_specs=[
            pl.BlockSpec((N_DRUG, blk), lambda j: (0, j)),
            pl.BlockSpec((N_TARGET, blk), lambda j: (0, j)),
        ],
        out_specs=[
            pl.BlockSpec((1, blk), lambda j: (0, j)),
            pl.BlockSpec((1, blk), lambda j: (0, j)),
            pl.BlockSpec((1, 2), lambda j: (0, 0), memory_space=pltpu.SMEM),
        ],
        out_shape=[
            jax.ShapeDtypeStruct((1, N_DRUG), jnp.float32),
            jax.ShapeDtypeStruct((1, N_TARGET), jnp.float32),
            jax.ShapeDtypeStruct((1, 2), jnp.float32),
        ],
        scratch_shapes=[pltpu.SMEM((2,), jnp.float32)],
    )(Kd, Kt)


def _final_body(kd_ref, kt_ref, a1_ref, a2_ref, dd_ref, dt_ref, mn_ref, o_ref):
    mnd = mn_ref[0, 0]
    mnt = mn_ref[0, 1]
    dd = dd_ref[...]
    dt = dt_ref[...]
    sd = 1.0 / jnp.where(dd == 0, mnd, dd)
    st = 1.0 / jnp.where(dt == 0, mnt, dt)
    kd = jnp.abs(kd_ref[...])
    kd = jnp.where(kd == 0, mnd, kd) * sd
    kt = jnp.abs(kt_ref[...])
    kt = jnp.where(kt == 0, mnt, kt) * st
    t1 = _dot(kd, a1_ref[...], prec=jax.lax.Precision.DEFAULT)
    t2 = lax.dot_general(a2_ref[...], kt, (((0,), (1,)), ((), ())),
                         precision=jax.lax.Precision.DEFAULT,
                         preferred_element_type=jnp.float32)
    o_ref[...] = 0.5 * (t1 + t2)


def _final(Kd, Kt, a1, a2, dd, dt, mn):
    blk = 256
    return pl.pallas_call(
        _final_body,
        grid=(N_DRUG // blk,),
        in_specs=[
            pl.BlockSpec((blk, N_DRUG), lambda i: (i, 0)),
            pl.BlockSpec((N_TARGET, N_TARGET), lambda i: (0, 0)),
            pl.BlockSpec((N_DRUG, N_TARGET), lambda i: (0, 0)),
            pl.BlockSpec((N_TARGET, blk), lambda i: (0, i)),
            pl.BlockSpec((1, N_DRUG), lambda i: (0, 0)),
            pl.BlockSpec((1, N_TARGET), lambda i: (0, 0)),
            pl.BlockSpec((1, 2), lambda i: (0, 0), memory_space=pltpu.SMEM),
        ],
        out_specs=pl.BlockSpec((blk, N_TARGET), lambda i: (i, 0)),
        out_shape=jax.ShapeDtypeStruct((N_DRUG, N_TARGET), jnp.float32),
    )(Kd, Kt, a1, a2, dd, dt, mn)


NWORK = 32
EW = EFULL // NWORK
CH = 96
NCHUNK = EW // CH
NGROUP = CH // 16


def _edge_body(h_hbm, sc_hbm, src_hbm, dst_hbm, dst2_hbm, z2_hbm,
               num_hbm,
               sc_tab, src1_v, dst1_v, dst2_v, ex0_v, ex1_v,
               rows0_v, rows1_v,
               num_sp, sg0, sg1, ss0, ss1):
    c = lax.axis_index("c")
    s = lax.axis_index("s")
    wid = s * 2 + c
    base = pl.multiple_of(wid * EW, 8)
    rows = (rows0_v, rows1_v)
    sg = (sg0, sg1)
    ss = (ss0, ss1)

    pltpu.sync_copy(sc_hbm, sc_tab)
    pltpu.sync_copy(src_hbm.at[pl.ds(base, EW)], src1_v)
    pltpu.sync_copy(dst_hbm.at[pl.ds(base, EW)], dst1_v)
    pltpu.sync_copy(dst2_hbm.at[wid], dst2_v)

    nsl = N // 16
    pltpu.sync_copy(z2_hbm.at[pl.ds(s * nsl, nsl)],
                    num_sp.at[pl.ds(s * nsl, nsl)])

    d_g0 = pltpu.async_copy(h_hbm.at[src1_v.at[pl.ds(0, CH)]], rows0_v, sg0)

    def group_body(g, carry2):
        go = g * 16
        sidx = src1_v[pl.ds(go, 16)]
        didx = dst1_v[pl.ds(go, 16)]
        as0 = plsc.load_gather(sc_tab, [sidx * 4])
        as1 = plsc.load_gather(sc_tab, [sidx * 4 + 1])
        ad0 = plsc.load_gather(sc_tab, [didx * 4 + 2])
        ad1 = plsc.load_gather(sc_tab, [didx * 4 + 3])
        e0 = as0 + ad0
        e1 = as1 + ad1
        e0 = jnp.where(e0 > 0, e0, 0.2 * e0)
        e1 = jnp.where(e1 > 0, e1, 0.2 * e1)
        ex0_v[pl.ds(go, 16)] = jnp.exp(e0)
        ex1_v[pl.ds(go, 16)] = jnp.exp(e1)
        return carry2

    lax.fori_loop(0, EW // 16, group_body, 0)
    plsc.subcore_barrier()
    iot = lax.iota(jnp.int32, 16)
    colD = jnp.full((16,), D, jnp.int32)

    d_s = [None] * NCHUNK
    d_g = [None] * NCHUNK
    d_g[0] = d_g0
    for k in range(NCHUNK):
        p = k & 1
        q = 1 - p
        rp, rq = rows[p], rows[q]
        if k + 1 < NCHUNK:
            if k >= 1:
                d_s[k - 1].wait()
            d_g[k + 1] = pltpu.async_copy(
                h_hbm.at[src1_v.at[pl.ds((k + 1) * CH, CH)]], rq, sg[q])
        d_g[k].wait()

        def den_body(g, carry2, rp=rp, k=k):
            go = g * 16
            eo = k * CH + go
            plsc.store_scatter(rp, [go + iot, colD], ex0_v[pl.ds(eo, 16)])
            plsc.store_scatter(rp, [go + iot, colD + 1],
                               ex1_v[pl.ds(eo, 16)])
            return carry2

        lax.fori_loop(0, NGROUP, den_body, 0)

        @plsc.parallel_loop(0, CH, step=1, unroll=8)
        def edge_body(e, rp=rp, k=k):
            i0 = jnp.full((16,), k * CH + e, jnp.int32)
            w0 = plsc.load_gather(ex0_v, [i0])
            w1 = plsc.load_gather(ex1_v, [i0])
            rp[e, pl.ds(0, 16)] = rp[e, pl.ds(0, 16)] * w0
            rp[e, pl.ds(16, 16)] = rp[e, pl.ds(16, 16)] * w0
            rp[e, pl.ds(32, 16)] = rp[e, pl.ds(32, 16)] * w1
            rp[e, pl.ds(48, 16)] = rp[e, pl.ds(48, 16)] * w1

        d_s[k] = pltpu.async_copy(rp, num_sp.at[dst2_v.at[k]], ss[p],
                                  add=True)

    d_s[NCHUNK - 2].wait()
    d_s[NCHUNK - 1].wait()
    plsc.subcore_barrier()

    pltpu.sync_copy(num_sp.at[pl.ds(s * nsl, nsl)],
                    num_hbm.at[c, pl.ds(s * nsl, nsl)])


def _edges_sc(h, scflat, srcfull, dstfull, dst2):
    mesh = plsc.VectorSubcoreMesh(core_axis_name="c", subcore_axis_name="s")
    z2 = jnp.zeros((N, DP), jnp.float32)
    call = pl.kernel(
        _edge_body,
        out_type=jax.ShapeDtypeStruct((2, N, DP), jnp.float32),
        mesh=mesh,
        compiler_params=pltpu.CompilerParams(needs_layout_passes=False),
        scratch_types=[
            pltpu.VMEM((4 * N,), jnp.float32),
            pltpu.VMEM((EW,), jnp.int32),
            pltpu.VMEM((EW,), jnp.int32),
            pltpu.VMEM((NCHUNK, CH), jnp.int32),
            pltpu.VMEM((EW,), jnp.float32),
            pltpu.VMEM((EW,), jnp.float32),
            pltpu.VMEM((CH, DP), jnp.float32),
            pltpu.VMEM((CH, DP), jnp.float32),
            pltpu.VMEM_SHARED((N, DP), jnp.float32),
            pltpu.SemaphoreType.DMA,
            pltpu.SemaphoreType.DMA,
            pltpu.SemaphoreType.DMA,
            pltpu.SemaphoreType.DMA,
        ],
    )
    return call(h, scflat, srcfull, dstfull, dst2, z2)


def kernel(x, Adj_data, edge_index, W1, a_src1, a_dst1, b1, W2, a_src2,
           a_dst2, b2, W3, a_src3, a_dst3, b3, W4, a_src4, a_dst4, b4,
           alpha1, alpha2, drug_sim, target_sim):
    del Adj_data

    loop = jnp.arange(N, dtype=edge_index.dtype)
    srcfull = jnp.concatenate([edge_index[0], loop])
    dstfull = jnp.concatenate([edge_index[1], loop])
    dst2 = dstfull.reshape(NWORK, NCHUNK, CH)

    Rsel = jnp.repeat(jnp.eye(HEADS, dtype=jnp.float32), F, axis=1)

    def make_aa(a_src, a_dst):
        z = jnp.zeros((F,), jnp.float32)
        c0 = jnp.concatenate([a_src[0], z])
        c1 = jnp.concatenate([z, a_src[1]])
        c2 = jnp.concatenate([a_dst[0], z])
        c3 = jnp.concatenate([z, a_dst[1]])
        return jnp.stack([c0, c1, c2, c3], axis=1)

    AAs = [make_aa(a_src1, a_dst1), make_aa(a_src2, a_dst2),
           make_aa(a_src3, a_dst3), make_aa(a_src4, a_dst4)]
    Ws = [W1, W2, W3, W4]
    bs = [b1.reshape(1, D), b2.reshape(1, D), b3.reshape(1, D),
          b4.reshape(1, D)]

    drug_ps = (1.0, 0.5, 0.333, 0.25)
    target_ps = (0.2, 0.2, 0.2, 0.2)

    h, sc = _proj1(x, W1, AAs[0])

    kd_acc = drug_sim
    kt_acc = target_sim
    dummyW = jnp.zeros((D, D), jnp.float32)
    dummyAA = jnp.zeros((D, 4), jnp.float32)
    for layer in range(4):
        num2 = _edges_sc(h, sc.reshape(-1), srcfull, dstfull, dst2)
        last = layer == 3
        Wn = Ws[layer + 1] if not last else dummyW
        AAn = AAs[layer + 1] if not last else dummyAA
        h, sc, ynd, ynt, mud, mut = _combine(
            num2, Rsel, bs[layer], Wn, AAn, project=not last)
        acc_scale = 0.2 if layer == 0 else 1.0
        kd_acc, kt_acc = _gip_accum(kd_acc, kt_acc, ynd, ynt, mud, mut,
                                    GAMMAS[layer], drug_ps[layer],
                                    target_ps[layer], acc_scale)

    dd, dt, mn = _diag_minpos(kd_acc, kt_acc)
    return _final(kd_acc, kt_acc, alpha1, alpha2, dd, dt, mn)


# speedup vs baseline: 132.0014x; 1.0268x over previous
import functools

import jax
import jax.numpy as jnp
from jax import lax
from jax.experimental import pallas as pl
from jax.experimental.pallas import tpu as pltpu
from jax.experimental.pallas import tpu_sc as plsc

N_DRUG = 1024
N_TARGET = 1024
N = N_DRUG + N_TARGET
E = 65536
EFULL = E + N
HEADS = 2
F = 32
D = HEADS * F
DP = 128
GAMMAS = (0.01, 0.01, 0.01, 0.01)

_P = jax.lax.Precision.HIGHEST


def _dot(a, b, dims=(((1,), (0,)), ((), ())), prec=_P):
    return lax.dot_general(a, b, dims, precision=prec,
                           preferred_element_type=jnp.float32)


def _proj1_body(x_ref, w_ref, aa_ref, h_ref, sc_ref):
    h = _dot(x_ref[...], w_ref[...], prec=jax.lax.Precision.DEFAULT)
    h_ref[...] = jnp.concatenate(
        [h, jnp.zeros((h.shape[0], DP - D), jnp.float32)], axis=1)
    sc_ref[...] = _dot(aa_ref[...], h, dims=(((0,), (1,)), ((), ())))


def _proj1(x, W1, AA):
    blk = 256
    return pl.pallas_call(
        _proj1_body,
        grid=(N // blk,),
        in_specs=[
            pl.BlockSpec((blk, N), lambda i: (i, 0)),
            pl.BlockSpec((N, D), lambda i: (0, 0)),
            pl.BlockSpec((D, 4), lambda i: (0, 0)),
        ],
        out_specs=[
            pl.BlockSpec((blk, DP), lambda i: (i, 0)),
            pl.BlockSpec((4, blk), lambda i: (0, i)),
        ],
        out_shape=[
            jax.ShapeDtypeStruct((N, DP), jnp.float32),
            jax.ShapeDtypeStruct((4, N), jnp.float32),
        ],
    )(x, W1, AA)


def _combine_body(num_ref, r_ref, b_ref, w_ref, aa_ref,
                  h_ref, sc_ref, ynd_ref, ynt_ref, mud_ref, mut_ref,
                  *, project):
    acc = num_ref[0] + num_ref[1]
    num = acc[:, :D]
    den = acc[:, D:D + HEADS]
    recip = 1.0 / (den + 1e-16)
    scale = _dot(recip, r_ref[...])
    H = jnp.maximum(num * scale + b_ref[...], 0.0)
    pad = jnp.zeros((N, DP - D), jnp.float32)
    if project:
        h = _dot(H, w_ref[...])
        h_ref[...] = jnp.concatenate([h, pad], axis=1)
        sc_ref[...] = _dot(aa_ref[...], h, dims=(((0,), (1,)), ((), ())))
    else:
        h_ref[...] = jnp.concatenate([H, pad], axis=1)
        sc_ref[...] = jnp.zeros_like(sc_ref)
    yd = H[:N_DRUG]
    yt = H[N_DRUG:]
    for y, yn_ref, mu_ref in ((yd, ynd_ref, mud_ref), (yt, ynt_ref, mut_ref)):
        mn = jnp.min(y, axis=1, keepdims=True)
        mx = jnp.max(y, axis=1, keepdims=True)
        yn = (y - mn) / (mx - mn + 1e-12)
        yn_ref[...] = yn
        mu_ref[...] = jnp.sum(yn * yn).reshape(1, 1) / y.shape[0]


def _combine(num2, Rsel, b_row, Wn, AAn, project):
    full = lambda shape: pl.BlockSpec(shape, lambda: tuple(0 for _ in shape))
    out_shapes = [
        jax.ShapeDtypeStruct((N, DP), jnp.float32),
        jax.ShapeDtypeStruct((4, N), jnp.float32),
        jax.ShapeDtypeStruct((N_DRUG, D), jnp.float32),
        jax.ShapeDtypeStruct((N_TARGET, D), jnp.float32),
        jax.ShapeDtypeStruct((1, 1), jnp.float32),
        jax.ShapeDtypeStruct((1, 1), jnp.float32),
    ]
    out_specs = [full((N, DP)), full((4, N)), full((N_DRUG, D)),
                 full((N_TARGET, D)), full((1, 1)), full((1, 1))]
    res = pl.pallas_call(
        functools.partial(_combine_body, project=project),
        in_specs=[full((2, N, DP)), full((HEADS, D)),
                  full((1, D)), full((D, D)), full((D, 4))],
        out_specs=out_specs,
        out_shape=out_shapes,
    )(num2, Rsel, b_row, Wn, AAn)
    return res


def _gip_half(acc, yt, ynf, mu, gamma, ps, acc_scale):
    kt = lax.dot_general(yt, ynf, (((1,), (1,)), ((), ())),
                         precision=jax.lax.Precision.DEFAULT,
                         preferred_element_type=jnp.float32)
    sqni = jnp.sum(yt * yt, axis=1, keepdims=True)
    ones = jnp.ones((1, D), jnp.float32)
    sqnj = lax.dot_general(ones, ynf * ynf, (((1,), (1,)), ((), ())),
                           precision=_P, preferred_element_type=jnp.float32)
    d = (sqni + sqnj - 2.0 * kt) / mu
    return acc * acc_scale + ps * jnp.exp(-d * gamma)


def _gip_body(accd_ref, acct_ref, ytd_ref, ynd_ref, ytt_ref, ynt_ref,
              mud_ref, mut_ref, od_ref, ot_ref, *, gamma, psd, pst,
              acc_scale):
    od_ref[...] = _gip_half(accd_ref[...], ytd_ref[...], ynd_ref[...],
                            mud_ref[0, 0], gamma, psd, acc_scale)
    ot_ref[...] = _gip_half(acct_ref[...], ytt_ref[...], ynt_ref[...],
                            mut_ref[0, 0], gamma, pst, acc_scale)


def _gip_accum(accd, acct, ynd, ynt, mud, mut, gamma, psd, pst, acc_scale):
    n = N_DRUG
    blk = 256
    return pl.pallas_call(
        functools.partial(_gip_body, gamma=gamma, psd=psd, pst=pst,
                          acc_scale=acc_scale),
        grid=(n // blk,),
        in_specs=[
            pl.BlockSpec((blk, n), lambda i: (i, 0)),
            pl.BlockSpec((blk, n), lambda i: (i, 0)),
            pl.BlockSpec((blk, D), lambda i: (i, 0)),
            pl.BlockSpec((n, D), lambda i: (0, 0)),
            pl.BlockSpec((blk, D), lambda i: (i, 0)),
            pl.BlockSpec((n, D), lambda i: (0, 0)),
            pl.BlockSpec((1, 1), lambda i: (0, 0)),
            pl.BlockSpec((1, 1), lambda i: (0, 0)),
        ],
        out_specs=[
            pl.BlockSpec((blk, n), lambda i: (i, 0)),
            pl.BlockSpec((blk, n), lambda i: (i, 0)),
        ],
        out_shape=[
            jax.ShapeDtypeStruct((n, n), jnp.float32),
            jax.ShapeDtypeStruct((n, n), jnp.float32),
        ],
        input_output_aliases={0: 0, 1: 1},
    )(accd, acct, ynd, ynd, ynt, ynt, mud, mut)


def _diag_body(kd_ref, kt_ref, dd_ref, dt_ref, mn_ref, mn_acc):
    j = pl.program_id(0)
    blk = kd_ref.shape[1]
    n = kd_ref.shape[0]
    rows = lax.broadcasted_iota(jnp.int32, (n, blk), 0)
    cols = lax.broadcasted_iota(jnp.int32, (n, blk), 1)
    mask = rows == cols + j * blk

    @pl.when(j == 0)
    def _init():
        mn_acc[0] = jnp.inf
        mn_acc[1] = jnp.inf

    for k_ref, d_ref, slot in ((kd_ref, dd_ref, 0), (kt_ref, dt_ref, 1)):
        k = jnp.abs(k_ref[...])
        d_ref[...] = jnp.sum(jnp.where(mask, k, 0.0), axis=0, keepdims=True)
        pos = jnp.where(k > 0, k, jnp.inf)
        mn_acc[slot] = jnp.minimum(mn_acc[slot], jnp.min(pos))
    mn_ref[0, 0] = mn_acc[0]
    mn_ref[0, 1] = mn_acc[1]


def _diag_minpos(Kd, Kt):
    blk = 256
    return pl.pallas_call(
        _diag_body,
        grid=(N_DRUG // blk,),
        in_specs=[
            pl.BlockSpec((N_DRUG, blk), lambda j: (0, j)),
            pl.BlockSpec((N_TARGET, blk), lambda j: (0, j)),
        ],
        out_specs=[
            pl.BlockSpec((1, blk), lambda j: (0, j)),
            pl.BlockSpec((1, blk), lambda j: (0, j)),
            pl.BlockSpec((1, 2), lambda j: (0, 0), memory_space=pltpu.SMEM),
        ],
        out_shape=[
            jax.ShapeDtypeStruct((1, N_DRUG), jnp.float32),
            jax.ShapeDtypeStruct((1, N_TARGET), jnp.float32),
            jax.ShapeDtypeStruct((1, 2), jnp.float32),
        ],
        scratch_shapes=[pltpu.SMEM((2,), jnp.float32)],
    )(Kd, Kt)


def _final_body(kd_ref, kt_ref, a1_ref, a2_ref, dd_ref, dt_ref, mn_ref, o_ref):
    mnd = mn_ref[0, 0]
    mnt = mn_ref[0, 1]
    dd = dd_ref[...]
    dt = dt_ref[...]
    sd = 1.0 / jnp.where(dd == 0, mnd, dd)
    st = 1.0 / jnp.where(dt == 0, mnt, dt)
    kd = jnp.abs(kd_ref[...])
    kd = jnp.where(kd == 0, mnd, kd) * sd
    kt = jnp.abs(kt_ref[...])
    kt = jnp.where(kt == 0, mnt, kt) * st
    t1 = _dot(kd, a1_ref[...], prec=jax.lax.Precision.DEFAULT)
    t2 = lax.dot_general(a2_ref[...], kt, (((0,), (1,)), ((), ())),
                         precision=jax.lax.Precision.DEFAULT,
                         preferred_element_type=jnp.float32)
    o_ref[...] = 0.5 * (t1 + t2)


def _final(Kd, Kt, a1, a2, dd, dt, mn):
    blk = 256
    return pl.pallas_call(
        _final_body,
        grid=(N_DRUG // blk,),
        in_specs=[
            pl.BlockSpec((blk, N_DRUG), lambda i: (i, 0)),
            pl.BlockSpec((N_TARGET, N_TARGET), lambda i: (0, 0)),
            pl.BlockSpec((N_DRUG, N_TARGET), lambda i: (0, 0)),
            pl.BlockSpec((N_TARGET, blk), lambda i: (0, i)),
            pl.BlockSpec((1, N_DRUG), lambda i: (0, 0)),
            pl.BlockSpec((1, N_TARGET), lambda i: (0, 0)),
            pl.BlockSpec((1, 2), lambda i: (0, 0), memory_space=pltpu.SMEM),
        ],
        out_specs=pl.BlockSpec((blk, N_TARGET), lambda i: (i, 0)),
        out_shape=jax.ShapeDtypeStruct((N_DRUG, N_TARGET), jnp.float32),
    )(Kd, Kt, a1, a2, dd, dt, mn)


NWORK = 32
EW = EFULL // NWORK
CH = 96
NCHUNK = EW // CH
NGROUP = CH // 16


def _edge_body(h_hbm, sc_hbm, src_hbm, dst_hbm, z2_hbm,
               num_hbm,
               sc_tab, src1_v, dst1_v, dst2_v, ex0_v, ex1_v,
               rows0_v, rows1_v,
               num_sp, sg0, sg1, ss0, ss1):
    c = lax.axis_index("c")
    s = lax.axis_index("s")
    wid = s * 2 + c
    base = pl.multiple_of(wid * EW, 8)
    rows = (rows0_v, rows1_v)
    sg = (sg0, sg1)
    ss = (ss0, ss1)

    pltpu.sync_copy(sc_hbm, sc_tab)
    pltpu.sync_copy(src_hbm.at[pl.ds(base, EW)], src1_v)
    pltpu.sync_copy(dst_hbm.at[pl.ds(base, EW)], dst1_v)

    nsl = N // 16
    pltpu.sync_copy(z2_hbm.at[pl.ds(s * nsl, nsl)],
                    num_sp.at[pl.ds(s * nsl, nsl)])

    d_g0 = pltpu.async_copy(h_hbm.at[src1_v.at[pl.ds(0, CH)]], rows0_v, sg0)

    row0 = jnp.full((16,), 0, jnp.int32)
    row1 = jnp.full((16,), 1, jnp.int32)
    row2 = jnp.full((16,), 2, jnp.int32)
    row3 = jnp.full((16,), 3, jnp.int32)

    def group_body(g, carry2):
        go = g * 16
        sidx = src1_v[pl.ds(go, 16)]
        didx = dst1_v[pl.ds(go, 16)]
        k = g // NGROUP
        dst2_v[k, pl.ds((g % NGROUP) * 16, 16)] = didx
        as0 = plsc.load_gather(sc_tab, [row0, sidx])
        as1 = plsc.load_gather(sc_tab, [row1, sidx])
        ad0 = plsc.load_gather(sc_tab, [row2, didx])
        ad1 = plsc.load_gather(sc_tab, [row3, didx])
        e0 = as0 + ad0
        e1 = as1 + ad1
        e0 = jnp.where(e0 > 0, e0, 0.2 * e0)
        e1 = jnp.where(e1 > 0, e1, 0.2 * e1)
        ex0_v[pl.ds(go, 16)] = jnp.exp(e0)
        ex1_v[pl.ds(go, 16)] = jnp.exp(e1)
        return carry2

    lax.fori_loop(0, EW // 16, group_body, 0)
    plsc.subcore_barrier()
    iot = lax.iota(jnp.int32, 16)
    colD = jnp.full((16,), D, jnp.int32)

    d_s = [None] * NCHUNK
    d_g = [None] * NCHUNK
    d_g[0] = d_g0
    for k in range(NCHUNK):
        p = k & 1
        q = 1 - p
        rp, rq = rows[p], rows[q]
        if k + 1 < NCHUNK:
            if k >= 1:
                d_s[k - 1].wait()
            d_g[k + 1] = pltpu.async_copy(
                h_hbm.at[src1_v.at[pl.ds((k + 1) * CH, CH)]], rq, sg[q])
        d_g[k].wait()

        def den_body(g, carry2, rp=rp, k=k):
            go = g * 16
            eo = k * CH + go
            plsc.store_scatter(rp, [go + iot, colD], ex0_v[pl.ds(eo, 16)])
            plsc.store_scatter(rp, [go + iot, colD + 1],
                               ex1_v[pl.ds(eo, 16)])
            return carry2

        lax.fori_loop(0, NGROUP, den_body, 0)

        @plsc.parallel_loop(0, CH, step=1, unroll=8)
        def edge_body(e, rp=rp, k=k):
            i0 = jnp.full((16,), k * CH + e, jnp.int32)
            w0 = plsc.load_gather(ex0_v, [i0])
            w1 = plsc.load_gather(ex1_v, [i0])
            rp[e, pl.ds(0, 16)] = rp[e, pl.ds(0, 16)] * w0
            rp[e, pl.ds(16, 16)] = rp[e, pl.ds(16, 16)] * w0
            rp[e, pl.ds(32, 16)] = rp[e, pl.ds(32, 16)] * w1
            rp[e, pl.ds(48, 16)] = rp[e, pl.ds(48, 16)] * w1

        d_s[k] = pltpu.async_copy(rp, num_sp.at[dst2_v.at[k]], ss[p],
                                  add=True)

    d_s[NCHUNK - 2].wait()
    d_s[NCHUNK - 1].wait()
    plsc.subcore_barrier()

    pltpu.sync_copy(num_sp.at[pl.ds(s * nsl, nsl)],
                    num_hbm.at[c, pl.ds(s * nsl, nsl)])


def _edges_sc(h, scT, srcfull, dstfull):
    mesh = plsc.VectorSubcoreMesh(core_axis_name="c", subcore_axis_name="s")
    z2 = jnp.zeros((N, DP), jnp.float32)
    call = pl.kernel(
        _edge_body,
        out_type=jax.ShapeDtypeStruct((2, N, DP), jnp.float32),
        mesh=mesh,
        compiler_params=pltpu.CompilerParams(needs_layout_passes=False),
        scratch_types=[
            pltpu.VMEM((4, N), jnp.float32),
            pltpu.VMEM((EW,), jnp.int32),
            pltpu.VMEM((EW,), jnp.int32),
            pltpu.VMEM((NCHUNK, CH), jnp.int32),
            pltpu.VMEM((EW,), jnp.float32),
            pltpu.VMEM((EW,), jnp.float32),
            pltpu.VMEM((CH, DP), jnp.float32),
            pltpu.VMEM((CH, DP), jnp.float32),
            pltpu.VMEM_SHARED((N, DP), jnp.float32),
            pltpu.SemaphoreType.DMA,
            pltpu.SemaphoreType.DMA,
            pltpu.SemaphoreType.DMA,
            pltpu.SemaphoreType.DMA,
        ],
    )
    return call(h, scT, srcfull, dstfull, z2)


def _mkidx_body(src_ref, dst_ref, srcf_ref, dstf_ref):
    iota = lax.broadcasted_iota(jnp.int32, (N,), 0)
    srcf_ref[pl.ds(0, E)] = src_ref[...].reshape(E)
    srcf_ref[pl.ds(E, N)] = iota
    dstf_ref[pl.ds(0, E)] = dst_ref[...].reshape(E)
    dstf_ref[pl.ds(E, N)] = iota


def _mkidx(edge_index):
    ei3 = edge_index.reshape(2, 1, E)
    return pl.pallas_call(
        _mkidx_body,
        grid=(1,),
        in_specs=[
            pl.BlockSpec((1, 1, E), lambda i: (0, 0, 0)),
            pl.BlockSpec((1, 1, E), lambda i: (1, 0, 0)),
        ],
        out_specs=[
            pl.BlockSpec((EFULL,), lambda i: (0,)),
            pl.BlockSpec((EFULL,), lambda i: (0,)),
        ],
        out_shape=[
            jax.ShapeDtypeStruct((EFULL,), jnp.int32),
            jax.ShapeDtypeStruct((EFULL,), jnp.int32),
        ],
    )(ei3, ei3)


def kernel(x, Adj_data, edge_index, W1, a_src1, a_dst1, b1, W2, a_src2,
           a_dst2, b2, W3, a_src3, a_dst3, b3, W4, a_src4, a_dst4, b4,
           alpha1, alpha2, drug_sim, target_sim):
    del Adj_data

    srcfull, dstfull = _mkidx(edge_index)

    Rsel = jnp.repeat(jnp.eye(HEADS, dtype=jnp.float32), F, axis=1)

    def make_aa(a_src, a_dst):
        z = jnp.zeros((F,), jnp.float32)
        c0 = jnp.concatenate([a_src[0], z])
        c1 = jnp.concatenate([z, a_src[1]])
        c2 = jnp.concatenate([a_dst[0], z])
        c3 = jnp.concatenate([z, a_dst[1]])
        return jnp.stack([c0, c1, c2, c3], axis=1)

    AAs = [make_aa(a_src1, a_dst1), make_aa(a_src2, a_dst2),
           make_aa(a_src3, a_dst3), make_aa(a_src4, a_dst4)]
    Ws = [W1, W2, W3, W4]
    bs = [b1.reshape(1, D), b2.reshape(1, D), b3.reshape(1, D),
          b4.reshape(1, D)]

    drug_ps = (1.0, 0.5, 0.333, 0.25)
    target_ps = (0.2, 0.2, 0.2, 0.2)

    h, sc = _proj1(x, W1, AAs[0])

    kd_acc = drug_sim
    kt_acc = target_sim
    dummyW = jnp.zeros((D, D), jnp.float32)
    dummyAA = jnp.zeros((D, 4), jnp.float32)
    for layer in range(4):
        num2 = _edges_sc(h, sc, srcfull, dstfull)
        last = layer == 3
        Wn = Ws[layer + 1] if not last else dummyW
        AAn = AAs[layer + 1] if not last else dummyAA
        h, sc, ynd, ynt, mud, mut = _combine(
            num2, Rsel, bs[layer], Wn, AAn, project=not last)
        acc_scale = 0.2 if layer == 0 else 1.0
        kd_acc, kt_acc = _gip_accum(kd_acc, kt_acc, ynd, ynt, mud, mut,
                                    GAMMAS[layer], drug_ps[layer],
                                    target_ps[layer], acc_scale)

    dd, dt, mn = _diag_minpos(kd_acc, kt_acc)
    return _final(kd_acc, kt_acc, alpha1, alpha2, dd, dt, mn)
